# Initial kernel scaffold; baseline (speedup 1.0000x reference)
import jax
import jax.numpy as jnp
from jax.experimental import pallas as pl


def kernel(x, time_steps, static, params):
    raise NotImplementedError("write your pallas kernel here")


# speedup vs baseline: 517.1633x; 517.1633x over previous
import math
import numpy as np
import jax
import jax.numpy as jnp
from jax.experimental import pallas as pl
from jax.experimental.pallas import tpu as pltpu

_PHYSIO = [(0, 7), (0, 10), (0, 6), (0, 4), (0, 8), (0, 11), (1, 7), (2, 9),
           (3, 4), (4, 10), (5, 6), (6, 10), (8, 9), (8, 10), (10, 11), (11, 12)]
_B, _T, _DYN, _STAT, _POS, _NODES = 512, 32, 13, 7, 32, 13
_HEADS, _GAT_H = 4, 32
_HID = 256
_BBLK = 16


def _adj_np():
    a = np.zeros((_NODES, _NODES), np.float32)
    for u, v in _PHYSIO:
        a[u, v] = 1.0
        a[v, u] = 1.0
    np.fill_diagonal(a, 1.0)
    return a


def _pe_np(d_model=_POS, max_len=5000):
    pos = np.arange(max_len, dtype=np.float32)[:, None]
    div = np.exp(np.arange(0, d_model, 2).astype(np.float32) * (-math.log(10000.0) / d_model))
    pe = np.zeros((max_len, d_model), np.float32)
    pe[:, 0::2] = np.sin(pos * div)
    pe[:, 1::2] = np.cos(pos * div)
    return pe


_ADJ = _adj_np()
_PE = _pe_np()

_S = _NODES * _HEADS
_P = _NODES * _NODES * _HEADS
_F = _NODES * 128
_SRC_EXP = np.kron(np.eye(_NODES, dtype=np.float32),
                   np.tile(np.eye(_HEADS, dtype=np.float32), (1, _NODES)))
_DST_EXP = np.tile(np.eye(_S, dtype=np.float32), (1, _NODES))
_RED = np.ascontiguousarray(_DST_EXP.T)
_MASKP = np.repeat(_ADJ.reshape(-1), _HEADS)[None, :].astype(np.float32)
_EXPC = np.kron(np.eye(_NODES, dtype=np.float32),
                np.repeat(np.eye(_HEADS, dtype=np.float32), 32, axis=1))
_E13 = np.kron(np.eye(_NODES, dtype=np.float32), np.ones((1, 128), np.float32))
_ONE128 = np.ascontiguousarray(_E13.T) / 128.0


def _fold_lin_bn(lin, bn):
    s = bn["gamma"] * jax.lax.rsqrt(bn["var"] + 1e-5)
    w = lin["w"].T * s[None, :]
    b = (lin["b"] - bn["mean"]) * s + bn["beta"]
    return w, b[None, :]


def _bn_fold(bn):
    s = bn["gamma"] * jax.lax.rsqrt(bn["var"] + 1e-5)
    c = bn["beta"] - bn["mean"] * s
    return s[None, :], c[None, :]


def _ln_in(v, g, b):
    m = v.mean(-1, keepdims=True)
    var = ((v - m) ** 2).mean(-1, keepdims=True)
    return (v - m) * jax.lax.rsqrt(var + 1e-5) * g + b


def _erf(x):
    ax = jnp.abs(x)
    t = 1.0 / (1.0 + 0.3275911 * ax)
    poly = t * (0.254829592 + t * (-0.284496736 + t * (1.421413741 + t * (-1.453152027 + t * 1.061405429))))
    y = 1.0 - poly * jnp.exp(-ax * ax)
    return jnp.sign(x) * y


def _gelu_exact(x):
    return 0.5 * x * (1.0 + _erf(x * 0.7071067811865476))


def _attn_agg(hb, asrcB, adstB, srcexp, dstexp, red, maskp, expc):
    a_s = jnp.dot(hb, asrcB, preferred_element_type=jnp.float32)
    a_d = jnp.dot(hb, adstB, preferred_element_type=jnp.float32)
    e = (jnp.dot(a_s, srcexp, preferred_element_type=jnp.float32)
         + jnp.dot(a_d, dstexp, preferred_element_type=jnp.float32))
    e = jnp.where(e >= 0, e, 0.2 * e)
    e = jnp.where(maskp > 0, e, -1e30)
    m = jnp.max(e, axis=-1, keepdims=True)
    ee = jnp.exp(e - m) * maskp
    den = jnp.dot(ee, red, preferred_element_type=jnp.float32)
    den = jnp.dot(den, dstexp, preferred_element_type=jnp.float32)
    alpha = ee / (den + 1e-16)
    acc = jnp.zeros_like(hb)
    for i in range(_NODES):
        aexp = jnp.dot(alpha[:, i * _S:(i + 1) * _S], expc,
                       preferred_element_type=jnp.float32)
        hs = hb[:, i * 128:(i + 1) * 128]
        acc = acc + aexp * jnp.concatenate([hs] * _NODES, axis=1)
    return acc


def _ln_big(v, one128, e13, gtile, btile):
    m13 = jnp.dot(v, one128, preferred_element_type=jnp.float32)
    d = v - jnp.dot(m13, e13, preferred_element_type=jnp.float32)
    v13 = jnp.dot(d * d, one128, preferred_element_type=jnp.float32)
    rstd = jax.lax.rsqrt(v13 + 1e-5)
    return d * jnp.dot(rstd, e13, preferred_element_type=jnp.float32) * gtile + btile


def _k1_body(x_ref, pos_ref, static_ref,
             wraw_ref, braw_ref, wstat_ref, bstat_ref,
             w0col_ref, w0pos_ref, pw0_ref, wppos_ref, pb_ref,
             asrc0_ref, adst0_ref, bias0_ref, ln0g_ref, ln0b_ref,
             w1t_ref, asrc1_ref, adst1_ref, bias1_ref, ln1g_ref, ln1b_ref,
             wg_ref, bg_ref,
             srcexp_ref, dstexp_ref, red_ref, maskp_ref, expc_ref,
             one128_ref, e13_ref,
             fused_out_ref, stat_out_ref):
    g = _BBLK * _T
    xb = x_ref[...].reshape(g, _DYN)
    posb = pos_ref[...].reshape(g, _POS)
    srcexp, dstexp, red = srcexp_ref[...], dstexp_ref[...], red_ref[...]
    maskp, expc = maskp_ref[...], expc_ref[...]
    one128, e13 = one128_ref[...], e13_ref[...]

    p0 = jnp.dot(posb, w0pos_ref[...], preferred_element_type=jnp.float32)
    w0col = w0col_ref[...]
    hbig = jnp.concatenate([xb[:, n:n + 1] * w0col + p0 for n in range(_NODES)], axis=1)
    rp = jnp.dot(posb, wppos_ref[...], preferred_element_type=jnp.float32) + pb_ref[...]
    pw0 = pw0_ref[...]
    resbig = jnp.concatenate([xb[:, n:n + 1] * pw0 + rp for n in range(_NODES)], axis=1)

    agg0 = _attn_agg(hbig, asrc0_ref[...], adst0_ref[...], srcexp, dstexp, red, maskp, expc)
    agg0 = agg0 + bias0_ref[...]
    h1 = jax.nn.relu(_ln_big(agg0, one128, e13, ln0g_ref[...], ln0b_ref[...]) + resbig)

    w1t = w1t_ref[...]
    h14 = jnp.concatenate(
        [jnp.dot(h1[:, n * 128:(n + 1) * 128], w1t, preferred_element_type=jnp.float32)
         for n in range(_NODES)], axis=1)
    agg1 = _attn_agg(h14, asrc1_ref[...], adst1_ref[...], srcexp, dstexp, red, maskp, expc)
    agg1 = agg1 + bias1_ref[...]
    h2 = jax.nn.relu(_ln_big(agg1, one128, e13, ln1g_ref[...], ln1b_ref[...]) + h1)

    gat_seq = h2[:, 0:128]
    for n in range(1, _NODES):
        gat_seq = gat_seq + h2[:, n * 128:(n + 1) * 128]
    gat_seq = gat_seq * (1.0 / _NODES)

    raw = jnp.dot(xb, wraw_ref[...], preferred_element_type=jnp.float32) + braw_ref[...]
    raw = _gelu_exact(raw)
    ff = jnp.concatenate([raw, gat_seq], axis=-1)
    gate = jax.nn.sigmoid(jnp.dot(ff, wg_ref[...], preferred_element_type=jnp.float32) + bg_ref[...])
    fused = gate * ff
    fused_out_ref[...] = fused.reshape(_BBLK, _T, 256).transpose(1, 0, 2)

    st = jnp.dot(static_ref[...], wstat_ref[...], preferred_element_type=jnp.float32) + bstat_ref[...]
    stat_out_ref[...] = jnp.where(st >= 0, st, 0.01 * st)


def _gru_step(xt, h, wih, whh, bih, bhh):
    gi = jnp.dot(xt, wih, preferred_element_type=jnp.float32) + bih
    gh = jnp.dot(h, whh, preferred_element_type=jnp.float32) + bhh
    r = jax.nn.sigmoid(gi[:, :_HID] + gh[:, :_HID])
    z = jax.nn.sigmoid(gi[:, _HID:2 * _HID] + gh[:, _HID:2 * _HID])
    n = jnp.tanh(gi[:, 2 * _HID:] + r * gh[:, 2 * _HID:])
    return (1.0 - z) * n + z * h


def _k2_body(fused_f_ref, fused_b_ref, stat_ref,
             wih_f_ref, whh_f_ref, bih_f_ref, bhh_f_ref,
             wih_b_ref, whh_b_ref, bih_b_ref, bhh_b_ref,
             f1_ref, b1_ref, hf_scr, hb_scr):
    t = pl.program_id(0)

    @pl.when(t == 0)
    def _():
        hf_scr[...] = jnp.zeros_like(hf_scr)
        hb_scr[...] = jnp.zeros_like(hb_scr)

    stat = stat_ref[...]
    xf = jnp.concatenate([fused_f_ref[0], stat], axis=-1)
    hf = _gru_step(xf, hf_scr[...], wih_f_ref[...], whh_f_ref[...], bih_f_ref[...], bhh_f_ref[...])
    hf_scr[...] = hf
    f1_ref[0] = hf

    xb = jnp.concatenate([fused_b_ref[0], stat], axis=-1)
    hb = _gru_step(xb, hb_scr[...], wih_b_ref[...], whh_b_ref[...], bih_b_ref[...], bhh_b_ref[...])
    hb_scr[...] = hb
    b1_ref[0] = hb


def _k3_body(f1_ref, b1_ref, f1last_ref, b1last_ref,
             wih_f_ref, whh_f_ref, bih_f_ref, bhh_f_ref,
             wih_b_ref, whh_b_ref, bih_b_ref, bhh_b_ref,
             lng_ref, lnb_ref, rbn_s_ref, rbn_c_ref,
             wres_ref, bres_ref, wsh_ref, bsh_ref,
             obn_s_ref, obn_c_ref, wout_ref, bout_ref,
             o_ref, hf_scr, hb_scr):
    t = pl.program_id(0)

    @pl.when(t == 0)
    def _():
        hf_scr[...] = jnp.zeros_like(hf_scr)
        xlast = jnp.concatenate([f1last_ref[0], b1last_ref[0]], axis=-1)
        gi = jnp.dot(xlast, wih_b_ref[...], preferred_element_type=jnp.float32) + bih_b_ref[...]
        bhh = bhh_b_ref[...]
        r = jax.nn.sigmoid(gi[:, :_HID] + bhh[:, :_HID])
        z = jax.nn.sigmoid(gi[:, _HID:2 * _HID] + bhh[:, _HID:2 * _HID])
        n = jnp.tanh(gi[:, 2 * _HID:] + r * bhh[:, 2 * _HID:])
        hb_scr[...] = (1.0 - z) * n

    xt = jnp.concatenate([f1_ref[0], b1_ref[0]], axis=-1)
    hf = _gru_step(xt, hf_scr[...], wih_f_ref[...], whh_f_ref[...], bih_f_ref[...], bhh_f_ref[...])
    hf_scr[...] = hf

    @pl.when(t == _T - 1)
    def _():
        feat = jnp.concatenate([hf, hb_scr[...]], axis=-1)
        feat = _ln_in(feat, lng_ref[...], lnb_ref[...])
        v = feat * rbn_s_ref[...] + rbn_c_ref[...]
        v = jax.nn.relu(v)
        o1 = (jnp.dot(v, wres_ref[...], preferred_element_type=jnp.float32) + bres_ref[...]
              + jnp.dot(feat, wsh_ref[...], preferred_element_type=jnp.float32) + bsh_ref[...])
        o1 = o1 * obn_s_ref[...] + obn_c_ref[...]
        o1 = jnp.where(o1 >= 0, o1, 0.01 * o1)
        o_ref[...] = jnp.dot(o1, wout_ref[...], preferred_element_type=jnp.float32) + bout_ref[...]


def _full(shape):
    nd = len(shape)
    return pl.BlockSpec(shape, lambda *a: (0,) * nd)


def kernel(x, time_steps, static, params):
    wraw, braw = _fold_lin_bn(params["raw_proj"]["lin"], params["raw_proj"]["bn"])
    wstat, bstat = _fold_lin_bn(params["static_net"]["lin"], params["static_net"]["bn"])

    g0 = params["gat"][0]
    w0 = g0["lin_w"]
    w0col = w0[:, 0][None, :]
    w0pos = w0[:, 1:].T
    pw = g0["proj"]["w"]
    pw0 = pw[:, 0][None, :]
    wppos = pw[:, 1:].T
    pb = g0["proj"]["b"][None, :]
    g1 = params["gat"][1]

    def _att_big(att):
        a1 = (jnp.eye(_HEADS, dtype=jnp.float32)[:, None, :] * att[:, :, None]).reshape(128, _HEADS)
        return jnp.kron(jnp.eye(_NODES, dtype=jnp.float32), a1)

    def _tile13(v):
        return jnp.tile(v, _NODES)[None, :]

    wg = params["fusion_gate"]["w"].T
    bg = params["fusion_gate"]["b"][None, :]

    gru1f, gru1b = params["gru"][0]
    gru2f, gru2b = params["gru"][1]

    lng = params["gru_norm"]["gamma"][None, :]
    lnb = params["gru_norm"]["beta"][None, :]
    rbn_s, rbn_c = _bn_fold(params["res"]["bn"])
    wres = params["res"]["lin"]["w"].T
    bres = params["res"]["lin"]["b"][None, :]
    wsh = params["res"]["short"]["w"].T
    bsh = params["res"]["short"]["b"][None, :]
    obn_s, obn_c = _bn_fold(params["out_bn"])
    wout = params["out_lin"]["w"].T
    bout = params["out_lin"]["b"][None, :]

    pos = jnp.asarray(_PE)[time_steps]

    nblk = _B // _BBLK
    k1_w = [wraw, braw, wstat, bstat,
            w0col, w0pos, pw0, wppos, pb,
            _att_big(g0["att_src"]), _att_big(g0["att_dst"]),
            _tile13(g0["bias"]), _tile13(g0["ln"]["gamma"]), _tile13(g0["ln"]["beta"]),
            g1["lin_w"].T,
            _att_big(g1["att_src"]), _att_big(g1["att_dst"]),
            _tile13(g1["bias"]), _tile13(g1["ln"]["gamma"]), _tile13(g1["ln"]["beta"]),
            wg, bg,
            jnp.asarray(_SRC_EXP), jnp.asarray(_DST_EXP), jnp.asarray(_RED),
            jnp.asarray(_MASKP), jnp.asarray(_EXPC),
            jnp.asarray(_ONE128), jnp.asarray(_E13)]
    fused, statf = pl.pallas_call(
        _k1_body,
        grid=(nblk,),
        in_specs=[
            pl.BlockSpec((_BBLK, _T, _DYN), lambda i: (i, 0, 0)),
            pl.BlockSpec((_BBLK, _T, _POS), lambda i: (i, 0, 0)),
            pl.BlockSpec((_BBLK, _STAT), lambda i: (i, 0)),
        ] + [_full(w.shape) for w in k1_w],
        out_specs=[
            pl.BlockSpec((_T, _BBLK, 256), lambda i: (0, i, 0)),
            pl.BlockSpec((_BBLK, 64), lambda i: (i, 0)),
        ],
        out_shape=[
            jax.ShapeDtypeStruct((_T, _B, 256), jnp.float32),
            jax.ShapeDtypeStruct((_B, 64), jnp.float32),
        ],
        compiler_params=pltpu.CompilerParams(dimension_semantics=("arbitrary",)),
    )(x, pos, static, *k1_w)

    k2_w = [gru1f["w_ih"].T, gru1f["w_hh"].T, gru1f["b_ih"][None, :], gru1f["b_hh"][None, :],
            gru1b["w_ih"].T, gru1b["w_hh"].T, gru1b["b_ih"][None, :], gru1b["b_hh"][None, :]]
    f1, b1 = pl.pallas_call(
        _k2_body,
        grid=(_T,),
        in_specs=[
            pl.BlockSpec((1, _B, 256), lambda t: (t, 0, 0)),
            pl.BlockSpec((1, _B, 256), lambda t: (_T - 1 - t, 0, 0)),
            _full((_B, 64)),
        ] + [_full(w.shape) for w in k2_w],
        out_specs=[
            pl.BlockSpec((1, _B, _HID), lambda t: (t, 0, 0)),
            pl.BlockSpec((1, _B, _HID), lambda t: (_T - 1 - t, 0, 0)),
        ],
        out_shape=[
            jax.ShapeDtypeStruct((_T, _B, _HID), jnp.float32),
            jax.ShapeDtypeStruct((_T, _B, _HID), jnp.float32),
        ],
        scratch_shapes=[pltpu.VMEM((_B, _HID), jnp.float32)] * 2,
        compiler_params=pltpu.CompilerParams(dimension_semantics=("arbitrary",)),
    )(fused, fused, statf, *k2_w)

    k3_w = [gru2f["w_ih"].T, gru2f["w_hh"].T, gru2f["b_ih"][None, :], gru2f["b_hh"][None, :],
            gru2b["w_ih"].T, gru2b["w_hh"].T, gru2b["b_ih"][None, :], gru2b["b_hh"][None, :],
            lng, lnb, rbn_s, rbn_c, wres, bres, wsh, bsh, obn_s, obn_c, wout, bout]
    o = pl.pallas_call(
        _k3_body,
        grid=(_T,),
        in_specs=[
            pl.BlockSpec((1, _B, _HID), lambda t: (t, 0, 0)),
            pl.BlockSpec((1, _B, _HID), lambda t: (t, 0, 0)),
            pl.BlockSpec((1, _B, _HID), lambda t: (_T - 1, 0, 0)),
            pl.BlockSpec((1, _B, _HID), lambda t: (_T - 1, 0, 0)),
        ] + [_full(w.shape) for w in k3_w],
        out_specs=pl.BlockSpec((_B, _DYN), lambda t: (0, 0)),
        out_shape=jax.ShapeDtypeStruct((_B, _DYN), jnp.float32),
        scratch_shapes=[pltpu.VMEM((_B, _HID), jnp.float32)] * 2,
        compiler_params=pltpu.CompilerParams(dimension_semantics=("arbitrary",)),
    )(f1, b1, f1, b1, *k3_w)

    return o.reshape(_B, 1, _NODES)


# speedup vs baseline: 537.0172x; 1.0384x over previous
import functools
import math
import numpy as np
import jax
import jax.numpy as jnp
from jax.experimental import pallas as pl
from jax.experimental.pallas import tpu as pltpu
from jax.experimental.pallas import tpu_sc as plsc

_PHYSIO = [(0, 7), (0, 10), (0, 6), (0, 4), (0, 8), (0, 11), (1, 7), (2, 9),
           (3, 4), (4, 10), (5, 6), (6, 10), (8, 9), (8, 10), (10, 11), (11, 12)]
_B, _T, _DYN, _STAT, _POS, _NODES = 512, 32, 13, 7, 32, 13
_HEADS, _GAT_H = 4, 32
_HID = 256
_BBLK = 16


def _adj_np():
    a = np.zeros((_NODES, _NODES), np.float32)
    for u, v in _PHYSIO:
        a[u, v] = 1.0
        a[v, u] = 1.0
    np.fill_diagonal(a, 1.0)
    return a


def _pe_np(d_model=_POS, max_len=5000):
    pos = np.arange(max_len, dtype=np.float32)[:, None]
    div = np.exp(np.arange(0, d_model, 2).astype(np.float32) * (-math.log(10000.0) / d_model))
    pe = np.zeros((max_len, d_model), np.float32)
    pe[:, 0::2] = np.sin(pos * div)
    pe[:, 1::2] = np.cos(pos * div)
    return pe


_ADJ = _adj_np()
_PE = _pe_np()

_S = _NODES * _HEADS
_P = _NODES * _NODES * _HEADS
_F = _NODES * 128
_SRC_EXP = np.kron(np.eye(_NODES, dtype=np.float32),
                   np.tile(np.eye(_HEADS, dtype=np.float32), (1, _NODES)))
_DST_EXP = np.tile(np.eye(_S, dtype=np.float32), (1, _NODES))
_RED = np.ascontiguousarray(_DST_EXP.T)
_MASKP = np.repeat(_ADJ.reshape(-1), _HEADS)[None, :].astype(np.float32)
_EXPC = np.kron(np.eye(_NODES, dtype=np.float32),
                np.repeat(np.eye(_HEADS, dtype=np.float32), 32, axis=1))
_E13 = np.kron(np.eye(_NODES, dtype=np.float32), np.ones((1, 128), np.float32))
_ONE128 = np.ascontiguousarray(_E13.T) / 128.0


def _pe_gather(table, idx_flat):
    info = plsc.get_sparse_core_info()
    nw = info.num_cores * info.num_subcores
    n = idx_flat.shape[0]
    b_per_w = n // nw
    d = table.shape[1]
    mesh = plsc.VectorSubcoreMesh(core_axis_name="c", subcore_axis_name="s")

    @functools.partial(
        pl.kernel, mesh=mesh,
        out_type=jax.ShapeDtypeStruct((n, d), jnp.float32),
        scratch_types=[
            pltpu.VMEM((b_per_w,), jnp.int32),
            pltpu.VMEM((b_per_w, d), jnp.float32),
            pltpu.SemaphoreType.DMA,
        ],
    )
    def k(table_hbm, idx_hbm, out_hbm, idx_v, rows_v, sem):
        wid = jax.lax.axis_index("s") * info.num_cores + jax.lax.axis_index("c")
        base = wid * b_per_w
        pltpu.sync_copy(idx_hbm.at[pl.ds(base, b_per_w)], idx_v)
        pltpu.async_copy(table_hbm.at[idx_v], rows_v, sem).wait()
        pltpu.sync_copy(rows_v, out_hbm.at[pl.ds(base, b_per_w)])

    return k(table, idx_flat)


def _fold_lin_bn(lin, bn):
    s = bn["gamma"] * jax.lax.rsqrt(bn["var"] + 1e-5)
    w = lin["w"].T * s[None, :]
    b = (lin["b"] - bn["mean"]) * s + bn["beta"]
    return w, b[None, :]


def _bn_fold(bn):
    s = bn["gamma"] * jax.lax.rsqrt(bn["var"] + 1e-5)
    c = bn["beta"] - bn["mean"] * s
    return s[None, :], c[None, :]


def _ln_in(v, g, b):
    m = v.mean(-1, keepdims=True)
    var = ((v - m) ** 2).mean(-1, keepdims=True)
    return (v - m) * jax.lax.rsqrt(var + 1e-5) * g + b


def _erf(x):
    ax = jnp.abs(x)
    t = 1.0 / (1.0 + 0.3275911 * ax)
    poly = t * (0.254829592 + t * (-0.284496736 + t * (1.421413741 + t * (-1.453152027 + t * 1.061405429))))
    y = 1.0 - poly * jnp.exp(-ax * ax)
    return jnp.sign(x) * y


def _gelu_exact(x):
    return 0.5 * x * (1.0 + _erf(x * 0.7071067811865476))


def _attn_agg(hb, asrcB, adstB, srcexp, dstexp, red, maskp, expc):
    a_s = jnp.dot(hb, asrcB, preferred_element_type=jnp.float32)
    a_d = jnp.dot(hb, adstB, preferred_element_type=jnp.float32)
    e = (jnp.dot(a_s, srcexp, preferred_element_type=jnp.float32)
         + jnp.dot(a_d, dstexp, preferred_element_type=jnp.float32))
    e = jnp.where(e >= 0, e, 0.2 * e)
    e = jnp.where(maskp > 0, e, -1e30)
    m = jnp.max(e, axis=-1, keepdims=True)
    ee = jnp.exp(e - m) * maskp
    den = jnp.dot(ee, red, preferred_element_type=jnp.float32)
    den = jnp.dot(den, dstexp, preferred_element_type=jnp.float32)
    alpha = ee / (den + 1e-16)
    acc = jnp.zeros_like(hb)
    for i in range(_NODES):
        aexp = jnp.dot(alpha[:, i * _S:(i + 1) * _S], expc,
                       preferred_element_type=jnp.float32)
        hs = hb[:, i * 128:(i + 1) * 128]
        acc = acc + aexp * jnp.concatenate([hs] * _NODES, axis=1)
    return acc


def _ln_big(v, one128, e13, gtile, btile):
    m13 = jnp.dot(v, one128, preferred_element_type=jnp.float32)
    d = v - jnp.dot(m13, e13, preferred_element_type=jnp.float32)
    v13 = jnp.dot(d * d, one128, preferred_element_type=jnp.float32)
    rstd = jax.lax.rsqrt(v13 + 1e-5)
    return d * jnp.dot(rstd, e13, preferred_element_type=jnp.float32) * gtile + btile


def _k1_body(x_ref, pos_ref, static_ref,
             wraw_ref, braw_ref, wstat_ref, bstat_ref,
             w0col_ref, w0pos_ref, pw0_ref, wppos_ref, pb_ref,
             asrc0_ref, adst0_ref, bias0_ref, ln0g_ref, ln0b_ref,
             w1t_ref, asrc1_ref, adst1_ref, bias1_ref, ln1g_ref, ln1b_ref,
             wg_ref, bg_ref,
             srcexp_ref, dstexp_ref, red_ref, maskp_ref, expc_ref,
             one128_ref, e13_ref,
             fused_out_ref, stat_out_ref):
    g = _BBLK * _T
    xb = x_ref[...].reshape(g, _DYN)
    posb = pos_ref[...].reshape(g, _POS)
    srcexp, dstexp, red = srcexp_ref[...], dstexp_ref[...], red_ref[...]
    maskp, expc = maskp_ref[...], expc_ref[...]
    one128, e13 = one128_ref[...], e13_ref[...]

    p0 = jnp.dot(posb, w0pos_ref[...], preferred_element_type=jnp.float32)
    w0col = w0col_ref[...]
    hbig = jnp.concatenate([xb[:, n:n + 1] * w0col + p0 for n in range(_NODES)], axis=1)
    rp = jnp.dot(posb, wppos_ref[...], preferred_element_type=jnp.float32) + pb_ref[...]
    pw0 = pw0_ref[...]
    resbig = jnp.concatenate([xb[:, n:n + 1] * pw0 + rp for n in range(_NODES)], axis=1)

    agg0 = _attn_agg(hbig, asrc0_ref[...], adst0_ref[...], srcexp, dstexp, red, maskp, expc)
    agg0 = agg0 + bias0_ref[...]
    h1 = jax.nn.relu(_ln_big(agg0, one128, e13, ln0g_ref[...], ln0b_ref[...]) + resbig)

    w1t = w1t_ref[...]
    h14 = jnp.concatenate(
        [jnp.dot(h1[:, n * 128:(n + 1) * 128], w1t, preferred_element_type=jnp.float32)
         for n in range(_NODES)], axis=1)
    agg1 = _attn_agg(h14, asrc1_ref[...], adst1_ref[...], srcexp, dstexp, red, maskp, expc)
    agg1 = agg1 + bias1_ref[...]
    h2 = jax.nn.relu(_ln_big(agg1, one128, e13, ln1g_ref[...], ln1b_ref[...]) + h1)

    gat_seq = h2[:, 0:128]
    for n in range(1, _NODES):
        gat_seq = gat_seq + h2[:, n * 128:(n + 1) * 128]
    gat_seq = gat_seq * (1.0 / _NODES)

    raw = jnp.dot(xb, wraw_ref[...], preferred_element_type=jnp.float32) + braw_ref[...]
    raw = _gelu_exact(raw)
    ff = jnp.concatenate([raw, gat_seq], axis=-1)
    gate = jax.nn.sigmoid(jnp.dot(ff, wg_ref[...], preferred_element_type=jnp.float32) + bg_ref[...])
    fused = gate * ff
    fused_out_ref[...] = fused.reshape(_BBLK, _T, 256).transpose(1, 0, 2)

    st = jnp.dot(static_ref[...], wstat_ref[...], preferred_element_type=jnp.float32) + bstat_ref[...]
    stat_out_ref[...] = jnp.where(st >= 0, st, 0.01 * st)


def _gru_step(xt, h, wih, whh, bih, bhh):
    gi = jnp.dot(xt, wih, preferred_element_type=jnp.float32) + bih
    gh = jnp.dot(h, whh, preferred_element_type=jnp.float32) + bhh
    r = jax.nn.sigmoid(gi[:, :_HID] + gh[:, :_HID])
    z = jax.nn.sigmoid(gi[:, _HID:2 * _HID] + gh[:, _HID:2 * _HID])
    n = jnp.tanh(gi[:, 2 * _HID:] + r * gh[:, 2 * _HID:])
    return (1.0 - z) * n + z * h


def _k2_body(fused_f_ref, fused_b_ref, stat_ref,
             wih_f_ref, whh_f_ref, bih_f_ref, bhh_f_ref,
             wih_b_ref, whh_b_ref, bih_b_ref, bhh_b_ref,
             f1_ref, b1_ref, hf_scr, hb_scr):
    t = pl.program_id(0)

    @pl.when(t == 0)
    def _():
        hf_scr[...] = jnp.zeros_like(hf_scr)
        hb_scr[...] = jnp.zeros_like(hb_scr)

    stat = stat_ref[...]
    xf = jnp.concatenate([fused_f_ref[0], stat], axis=-1)
    hf = _gru_step(xf, hf_scr[...], wih_f_ref[...], whh_f_ref[...], bih_f_ref[...], bhh_f_ref[...])
    hf_scr[...] = hf
    f1_ref[0] = hf

    xb = jnp.concatenate([fused_b_ref[0], stat], axis=-1)
    hb = _gru_step(xb, hb_scr[...], wih_b_ref[...], whh_b_ref[...], bih_b_ref[...], bhh_b_ref[...])
    hb_scr[...] = hb
    b1_ref[0] = hb


def _k3_body(f1_ref, b1_ref, f1last_ref, b1last_ref,
             wih_f_ref, whh_f_ref, bih_f_ref, bhh_f_ref,
             wih_b_ref, whh_b_ref, bih_b_ref, bhh_b_ref,
             lng_ref, lnb_ref, rbn_s_ref, rbn_c_ref,
             wres_ref, bres_ref, wsh_ref, bsh_ref,
             obn_s_ref, obn_c_ref, wout_ref, bout_ref,
             o_ref, hf_scr, hb_scr):
    t = pl.program_id(0)

    @pl.when(t == 0)
    def _():
        hf_scr[...] = jnp.zeros_like(hf_scr)
        xlast = jnp.concatenate([f1last_ref[0], b1last_ref[0]], axis=-1)
        gi = jnp.dot(xlast, wih_b_ref[...], preferred_element_type=jnp.float32) + bih_b_ref[...]
        bhh = bhh_b_ref[...]
        r = jax.nn.sigmoid(gi[:, :_HID] + bhh[:, :_HID])
        z = jax.nn.sigmoid(gi[:, _HID:2 * _HID] + bhh[:, _HID:2 * _HID])
        n = jnp.tanh(gi[:, 2 * _HID:] + r * bhh[:, 2 * _HID:])
        hb_scr[...] = (1.0 - z) * n

    xt = jnp.concatenate([f1_ref[0], b1_ref[0]], axis=-1)
    hf = _gru_step(xt, hf_scr[...], wih_f_ref[...], whh_f_ref[...], bih_f_ref[...], bhh_f_ref[...])
    hf_scr[...] = hf

    @pl.when(t == _T - 1)
    def _():
        feat = jnp.concatenate([hf, hb_scr[...]], axis=-1)
        feat = _ln_in(feat, lng_ref[...], lnb_ref[...])
        v = feat * rbn_s_ref[...] + rbn_c_ref[...]
        v = jax.nn.relu(v)
        o1 = (jnp.dot(v, wres_ref[...], preferred_element_type=jnp.float32) + bres_ref[...]
              + jnp.dot(feat, wsh_ref[...], preferred_element_type=jnp.float32) + bsh_ref[...])
        o1 = o1 * obn_s_ref[...] + obn_c_ref[...]
        o1 = jnp.where(o1 >= 0, o1, 0.01 * o1)
        o_ref[...] = jnp.dot(o1, wout_ref[...], preferred_element_type=jnp.float32) + bout_ref[...]


def _full(shape):
    nd = len(shape)
    return pl.BlockSpec(shape, lambda *a: (0,) * nd)


def kernel(x, time_steps, static, params):
    wraw, braw = _fold_lin_bn(params["raw_proj"]["lin"], params["raw_proj"]["bn"])
    wstat, bstat = _fold_lin_bn(params["static_net"]["lin"], params["static_net"]["bn"])

    g0 = params["gat"][0]
    w0 = g0["lin_w"]
    w0col = w0[:, 0][None, :]
    w0pos = w0[:, 1:].T
    pw = g0["proj"]["w"]
    pw0 = pw[:, 0][None, :]
    wppos = pw[:, 1:].T
    pb = g0["proj"]["b"][None, :]
    g1 = params["gat"][1]

    def _att_big(att):
        a1 = (jnp.eye(_HEADS, dtype=jnp.float32)[:, None, :] * att[:, :, None]).reshape(128, _HEADS)
        return jnp.kron(jnp.eye(_NODES, dtype=jnp.float32), a1)

    def _tile13(v):
        return jnp.tile(v, _NODES)[None, :]

    wg = params["fusion_gate"]["w"].T
    bg = params["fusion_gate"]["b"][None, :]

    gru1f, gru1b = params["gru"][0]
    gru2f, gru2b = params["gru"][1]

    lng = params["gru_norm"]["gamma"][None, :]
    lnb = params["gru_norm"]["beta"][None, :]
    rbn_s, rbn_c = _bn_fold(params["res"]["bn"])
    wres = params["res"]["lin"]["w"].T
    bres = params["res"]["lin"]["b"][None, :]
    wsh = params["res"]["short"]["w"].T
    bsh = params["res"]["short"]["b"][None, :]
    obn_s, obn_c = _bn_fold(params["out_bn"])
    wout = params["out_lin"]["w"].T
    bout = params["out_lin"]["b"][None, :]

    idx = time_steps.reshape(-1).astype(jnp.int32)
    pe_pad = jnp.zeros((_PE.shape[0], 128), jnp.float32).at[:, :_POS].set(jnp.asarray(_PE))
    pos = _pe_gather(pe_pad, idx)[:, :_POS].reshape(_B, _T, _POS)

    nblk = _B // _BBLK
    k1_w = [wraw, braw, wstat, bstat,
            w0col, w0pos, pw0, wppos, pb,
            _att_big(g0["att_src"]), _att_big(g0["att_dst"]),
            _tile13(g0["bias"]), _tile13(g0["ln"]["gamma"]), _tile13(g0["ln"]["beta"]),
            g1["lin_w"].T,
            _att_big(g1["att_src"]), _att_big(g1["att_dst"]),
            _tile13(g1["bias"]), _tile13(g1["ln"]["gamma"]), _tile13(g1["ln"]["beta"]),
            wg, bg,
            jnp.asarray(_SRC_EXP), jnp.asarray(_DST_EXP), jnp.asarray(_RED),
            jnp.asarray(_MASKP), jnp.asarray(_EXPC),
            jnp.asarray(_ONE128), jnp.asarray(_E13)]
    fused, statf = pl.pallas_call(
        _k1_body,
        grid=(nblk,),
        in_specs=[
            pl.BlockSpec((_BBLK, _T, _DYN), lambda i: (i, 0, 0)),
            pl.BlockSpec((_BBLK, _T, _POS), lambda i: (i, 0, 0)),
            pl.BlockSpec((_BBLK, _STAT), lambda i: (i, 0)),
        ] + [_full(w.shape) for w in k1_w],
        out_specs=[
            pl.BlockSpec((_T, _BBLK, 256), lambda i: (0, i, 0)),
            pl.BlockSpec((_BBLK, 64), lambda i: (i, 0)),
        ],
        out_shape=[
            jax.ShapeDtypeStruct((_T, _B, 256), jnp.float32),
            jax.ShapeDtypeStruct((_B, 64), jnp.float32),
        ],
        compiler_params=pltpu.CompilerParams(dimension_semantics=("arbitrary",)),
    )(x, pos, static, *k1_w)

    k2_w = [gru1f["w_ih"].T, gru1f["w_hh"].T, gru1f["b_ih"][None, :], gru1f["b_hh"][None, :],
            gru1b["w_ih"].T, gru1b["w_hh"].T, gru1b["b_ih"][None, :], gru1b["b_hh"][None, :]]
    f1, b1 = pl.pallas_call(
        _k2_body,
        grid=(_T,),
        in_specs=[
            pl.BlockSpec((1, _B, 256), lambda t: (t, 0, 0)),
            pl.BlockSpec((1, _B, 256), lambda t: (_T - 1 - t, 0, 0)),
            _full((_B, 64)),
        ] + [_full(w.shape) for w in k2_w],
        out_specs=[
            pl.BlockSpec((1, _B, _HID), lambda t: (t, 0, 0)),
            pl.BlockSpec((1, _B, _HID), lambda t: (_T - 1 - t, 0, 0)),
        ],
        out_shape=[
            jax.ShapeDtypeStruct((_T, _B, _HID), jnp.float32),
            jax.ShapeDtypeStruct((_T, _B, _HID), jnp.float32),
        ],
        scratch_shapes=[pltpu.VMEM((_B, _HID), jnp.float32)] * 2,
        compiler_params=pltpu.CompilerParams(dimension_semantics=("arbitrary",)),
    )(fused, fused, statf, *k2_w)

    k3_w = [gru2f["w_ih"].T, gru2f["w_hh"].T, gru2f["b_ih"][None, :], gru2f["b_hh"][None, :],
            gru2b["w_ih"].T, gru2b["w_hh"].T, gru2b["b_ih"][None, :], gru2b["b_hh"][None, :],
            lng, lnb, rbn_s, rbn_c, wres, bres, wsh, bsh, obn_s, obn_c, wout, bout]
    o = pl.pallas_call(
        _k3_body,
        grid=(_T,),
        in_specs=[
            pl.BlockSpec((1, _B, _HID), lambda t: (t, 0, 0)),
            pl.BlockSpec((1, _B, _HID), lambda t: (t, 0, 0)),
            pl.BlockSpec((1, _B, _HID), lambda t: (_T - 1, 0, 0)),
            pl.BlockSpec((1, _B, _HID), lambda t: (_T - 1, 0, 0)),
        ] + [_full(w.shape) for w in k3_w],
        out_specs=pl.BlockSpec((_B, _DYN), lambda t: (0, 0)),
        out_shape=jax.ShapeDtypeStruct((_B, _DYN), jnp.float32),
        scratch_shapes=[pltpu.VMEM((_B, _HID), jnp.float32)] * 2,
        compiler_params=pltpu.CompilerParams(dimension_semantics=("arbitrary",)),
    )(f1, b1, f1, b1, *k3_w)

    return o.reshape(_B, 1, _NODES)


# speedup vs baseline: 751.8573x; 1.4001x over previous
import functools
import math
import numpy as np
import jax
import jax.numpy as jnp
from jax.experimental import pallas as pl
from jax.experimental.pallas import tpu as pltpu
from jax.experimental.pallas import tpu_sc as plsc

_PHYSIO = [(0, 7), (0, 10), (0, 6), (0, 4), (0, 8), (0, 11), (1, 7), (2, 9),
           (3, 4), (4, 10), (5, 6), (6, 10), (8, 9), (8, 10), (10, 11), (11, 12)]
_B, _T, _DYN, _STAT, _POS, _NODES = 512, 32, 13, 7, 32, 13
_HEADS, _GAT_H = 4, 32
_HID = 256
_BBLK = 16


def _adj_np():
    a = np.zeros((_NODES, _NODES), np.float32)
    for u, v in _PHYSIO:
        a[u, v] = 1.0
        a[v, u] = 1.0
    np.fill_diagonal(a, 1.0)
    return a


def _pe_np(d_model=_POS, max_len=5000):
    pos = np.arange(max_len, dtype=np.float32)[:, None]
    div = np.exp(np.arange(0, d_model, 2).astype(np.float32) * (-math.log(10000.0) / d_model))
    pe = np.zeros((max_len, d_model), np.float32)
    pe[:, 0::2] = np.sin(pos * div)
    pe[:, 1::2] = np.cos(pos * div)
    return pe


_ADJ = _adj_np()
_PE = _pe_np()

_S = _NODES * _HEADS
_P = _NODES * _NODES * _HEADS
_F = _NODES * 128
_SRC_EXP = np.kron(np.eye(_NODES, dtype=np.float32),
                   np.tile(np.eye(_HEADS, dtype=np.float32), (1, _NODES)))
_DST_EXP = np.tile(np.eye(_S, dtype=np.float32), (1, _NODES))
_RED = np.ascontiguousarray(_DST_EXP.T)
_MASKP = np.repeat(_ADJ.reshape(-1), _HEADS)[None, :].astype(np.float32)
_EXPC = np.kron(np.eye(_NODES, dtype=np.float32),
                np.repeat(np.eye(_HEADS, dtype=np.float32), 32, axis=1))
_NBRS = [[j for j in range(_NODES) if _ADJ[i, j] > 0] for i in range(_NODES)]
_EXPCI = [np.concatenate([_EXPC[:, j * 128:(j + 1) * 128] for j in _NBRS[i]], axis=1)
          for i in range(_NODES)]
_E13 = np.kron(np.eye(_NODES, dtype=np.float32), np.ones((1, 128), np.float32))
_ONE128 = np.ascontiguousarray(_E13.T) / 128.0


def _pe_gather(table, idx_flat):
    info = plsc.get_sparse_core_info()
    nw = info.num_cores * info.num_subcores
    n = idx_flat.shape[0]
    b_per_w = n // nw
    d = table.shape[1]
    mesh = plsc.VectorSubcoreMesh(core_axis_name="c", subcore_axis_name="s")

    @functools.partial(
        pl.kernel, mesh=mesh,
        out_type=jax.ShapeDtypeStruct((n, d), jnp.float32),
        scratch_types=[
            pltpu.VMEM((b_per_w,), jnp.int32),
            pltpu.VMEM((b_per_w, d), jnp.float32),
            pltpu.SemaphoreType.DMA,
        ],
    )
    def k(table_hbm, idx_hbm, out_hbm, idx_v, rows_v, sem):
        wid = jax.lax.axis_index("s") * info.num_cores + jax.lax.axis_index("c")
        base = wid * b_per_w
        pltpu.sync_copy(idx_hbm.at[pl.ds(base, b_per_w)], idx_v)
        pltpu.async_copy(table_hbm.at[idx_v], rows_v, sem).wait()
        pltpu.sync_copy(rows_v, out_hbm.at[pl.ds(base, b_per_w)])

    return k(table, idx_flat)


def _fold_lin_bn(lin, bn):
    s = bn["gamma"] * jax.lax.rsqrt(bn["var"] + 1e-5)
    w = lin["w"].T * s[None, :]
    b = (lin["b"] - bn["mean"]) * s + bn["beta"]
    return w, b[None, :]


def _bn_fold(bn):
    s = bn["gamma"] * jax.lax.rsqrt(bn["var"] + 1e-5)
    c = bn["beta"] - bn["mean"] * s
    return s[None, :], c[None, :]


def _ln_in(v, g, b):
    m = v.mean(-1, keepdims=True)
    var = ((v - m) ** 2).mean(-1, keepdims=True)
    return (v - m) * jax.lax.rsqrt(var + 1e-5) * g + b


def _erf(x):
    ax = jnp.abs(x)
    t = 1.0 / (1.0 + 0.3275911 * ax)
    poly = t * (0.254829592 + t * (-0.284496736 + t * (1.421413741 + t * (-1.453152027 + t * 1.061405429))))
    y = 1.0 - poly * jnp.exp(-ax * ax)
    return jnp.sign(x) * y


def _gelu_exact(x):
    return 0.5 * x * (1.0 + _erf(x * 0.7071067811865476))


def _attn_agg(hb, asrcB, adstB, srcexp, dstexp, red, maskp, expc, expcis):
    a_s = jnp.dot(hb, asrcB, preferred_element_type=jnp.float32)
    a_d = jnp.dot(hb, adstB, preferred_element_type=jnp.float32)
    e = (jnp.dot(a_s, srcexp, preferred_element_type=jnp.float32)
         + jnp.dot(a_d, dstexp, preferred_element_type=jnp.float32))
    e = jnp.where(e >= 0, e, 0.2 * e)
    e = jnp.where(maskp > 0, e, -1e30)
    m = jnp.max(e, axis=-1, keepdims=True)
    ee = jnp.exp(e - m) * maskp
    den = jnp.dot(ee, red, preferred_element_type=jnp.float32)
    denrec = 1.0 / (den + 1e-16)
    parts = [None] * _NODES
    for i in range(_NODES):
        aexp = jnp.dot(ee[:, i * _S:(i + 1) * _S], expcis[i],
                       preferred_element_type=jnp.float32)
        hs = hb[:, i * 128:(i + 1) * 128]
        for k, j in enumerate(_NBRS[i]):
            c = aexp[:, k * 128:(k + 1) * 128] * hs
            parts[j] = c if parts[j] is None else parts[j] + c
    acc = jnp.concatenate(parts, axis=1)
    return acc * jnp.dot(denrec, expc, preferred_element_type=jnp.float32)


def _ln_big(v, one128, e13, gtile, btile):
    m13 = jnp.dot(v, one128, preferred_element_type=jnp.float32)
    d = v - jnp.dot(m13, e13, preferred_element_type=jnp.float32)
    v13 = jnp.dot(d * d, one128, preferred_element_type=jnp.float32)
    rstd = jax.lax.rsqrt(v13 + 1e-5)
    return d * jnp.dot(rstd, e13, preferred_element_type=jnp.float32) * gtile + btile


def _k1_body(x_ref, pos_ref, static_ref,
             wraw_ref, braw_ref, wstat_ref, bstat_ref,
             w0col_ref, w0pos_ref, pw0_ref, wppos_ref, pb_ref,
             asrc0_ref, adst0_ref, bias0_ref, ln0g_ref, ln0b_ref,
             w1t_ref, asrc1_ref, adst1_ref, bias1_ref, ln1g_ref, ln1b_ref,
             wg_ref, bg_ref,
             srcexp_ref, dstexp_ref, red_ref, maskp_ref, expc_ref,
             one128_ref, e13_ref,
             *rest):
    expci = [r[...] for r in rest[:_NODES]]
    fused_out_ref, stat_out_ref = rest[_NODES], rest[_NODES + 1]
    g = _BBLK * _T
    xb = x_ref[...].reshape(g, _DYN)
    posb = pos_ref[...].reshape(g, _POS)
    srcexp, dstexp, red = srcexp_ref[...], dstexp_ref[...], red_ref[...]
    maskp, expc = maskp_ref[...], expc_ref[...]
    one128, e13 = one128_ref[...], e13_ref[...]

    p0 = jnp.dot(posb, w0pos_ref[...], preferred_element_type=jnp.float32)
    w0col = w0col_ref[...]
    hbig = jnp.concatenate([xb[:, n:n + 1] * w0col + p0 for n in range(_NODES)], axis=1)
    rp = jnp.dot(posb, wppos_ref[...], preferred_element_type=jnp.float32) + pb_ref[...]
    pw0 = pw0_ref[...]
    resbig = jnp.concatenate([xb[:, n:n + 1] * pw0 + rp for n in range(_NODES)], axis=1)

    agg0 = _attn_agg(hbig, asrc0_ref[...], adst0_ref[...], srcexp, dstexp, red, maskp, expc, expci)
    agg0 = agg0 + bias0_ref[...]
    h1 = jax.nn.relu(_ln_big(agg0, one128, e13, ln0g_ref[...], ln0b_ref[...]) + resbig)

    w1t = w1t_ref[...]
    h14 = jnp.concatenate(
        [jnp.dot(h1[:, n * 128:(n + 1) * 128], w1t, preferred_element_type=jnp.float32)
         for n in range(_NODES)], axis=1)
    agg1 = _attn_agg(h14, asrc1_ref[...], adst1_ref[...], srcexp, dstexp, red, maskp, expc, expci)
    agg1 = agg1 + bias1_ref[...]
    h2 = jax.nn.relu(_ln_big(agg1, one128, e13, ln1g_ref[...], ln1b_ref[...]) + h1)

    gat_seq = h2[:, 0:128]
    for n in range(1, _NODES):
        gat_seq = gat_seq + h2[:, n * 128:(n + 1) * 128]
    gat_seq = gat_seq * (1.0 / _NODES)

    raw = jnp.dot(xb, wraw_ref[...], preferred_element_type=jnp.float32) + braw_ref[...]
    raw = _gelu_exact(raw)
    ff = jnp.concatenate([raw, gat_seq], axis=-1)
    gate = jax.nn.sigmoid(jnp.dot(ff, wg_ref[...], preferred_element_type=jnp.float32) + bg_ref[...])
    fused = gate * ff
    fused_out_ref[...] = fused.reshape(_BBLK, _T, 256).transpose(1, 0, 2)

    st = jnp.dot(static_ref[...], wstat_ref[...], preferred_element_type=jnp.float32) + bstat_ref[...]
    stat_out_ref[...] = jnp.where(st >= 0, st, 0.01 * st)


def _gru_step(xt, h, wih, whh, bih, bhh):
    gi = jnp.dot(xt, wih, preferred_element_type=jnp.float32) + bih
    gh = jnp.dot(h, whh, preferred_element_type=jnp.float32) + bhh
    r = jax.nn.sigmoid(gi[:, :_HID] + gh[:, :_HID])
    z = jax.nn.sigmoid(gi[:, _HID:2 * _HID] + gh[:, _HID:2 * _HID])
    n = jnp.tanh(gi[:, 2 * _HID:] + r * gh[:, 2 * _HID:])
    return (1.0 - z) * n + z * h


def _k2_body(fused_f_ref, fused_b_ref, stat_ref,
             wih_f_ref, whh_f_ref, bih_f_ref, bhh_f_ref,
             wih_b_ref, whh_b_ref, bih_b_ref, bhh_b_ref,
             f1_ref, b1_ref, hf_scr, hb_scr):
    t = pl.program_id(0)

    @pl.when(t == 0)
    def _():
        hf_scr[...] = jnp.zeros_like(hf_scr)
        hb_scr[...] = jnp.zeros_like(hb_scr)

    stat = stat_ref[...]
    xf = jnp.concatenate([fused_f_ref[0], stat], axis=-1)
    hf = _gru_step(xf, hf_scr[...], wih_f_ref[...], whh_f_ref[...], bih_f_ref[...], bhh_f_ref[...])
    hf_scr[...] = hf
    f1_ref[0] = hf

    xb = jnp.concatenate([fused_b_ref[0], stat], axis=-1)
    hb = _gru_step(xb, hb_scr[...], wih_b_ref[...], whh_b_ref[...], bih_b_ref[...], bhh_b_ref[...])
    hb_scr[...] = hb
    b1_ref[0] = hb


def _k3_body(f1_ref, b1_ref, f1last_ref, b1last_ref,
             wih_f_ref, whh_f_ref, bih_f_ref, bhh_f_ref,
             wih_b_ref, whh_b_ref, bih_b_ref, bhh_b_ref,
             lng_ref, lnb_ref, rbn_s_ref, rbn_c_ref,
             wres_ref, bres_ref, wsh_ref, bsh_ref,
             obn_s_ref, obn_c_ref, wout_ref, bout_ref,
             o_ref, hf_scr, hb_scr):
    t = pl.program_id(0)

    @pl.when(t == 0)
    def _():
        hf_scr[...] = jnp.zeros_like(hf_scr)
        xlast = jnp.concatenate([f1last_ref[0], b1last_ref[0]], axis=-1)
        gi = jnp.dot(xlast, wih_b_ref[...], preferred_element_type=jnp.float32) + bih_b_ref[...]
        bhh = bhh_b_ref[...]
        r = jax.nn.sigmoid(gi[:, :_HID] + bhh[:, :_HID])
        z = jax.nn.sigmoid(gi[:, _HID:2 * _HID] + bhh[:, _HID:2 * _HID])
        n = jnp.tanh(gi[:, 2 * _HID:] + r * bhh[:, 2 * _HID:])
        hb_scr[...] = (1.0 - z) * n

    xt = jnp.concatenate([f1_ref[0], b1_ref[0]], axis=-1)
    hf = _gru_step(xt, hf_scr[...], wih_f_ref[...], whh_f_ref[...], bih_f_ref[...], bhh_f_ref[...])
    hf_scr[...] = hf

    @pl.when(t == _T - 1)
    def _():
        feat = jnp.concatenate([hf, hb_scr[...]], axis=-1)
        feat = _ln_in(feat, lng_ref[...], lnb_ref[...])
        v = feat * rbn_s_ref[...] + rbn_c_ref[...]
        v = jax.nn.relu(v)
        o1 = (jnp.dot(v, wres_ref[...], preferred_element_type=jnp.float32) + bres_ref[...]
              + jnp.dot(feat, wsh_ref[...], preferred_element_type=jnp.float32) + bsh_ref[...])
        o1 = o1 * obn_s_ref[...] + obn_c_ref[...]
        o1 = jnp.where(o1 >= 0, o1, 0.01 * o1)
        o_ref[...] = jnp.dot(o1, wout_ref[...], preferred_element_type=jnp.float32) + bout_ref[...]


def _full(shape):
    nd = len(shape)
    return pl.BlockSpec(shape, lambda *a: (0,) * nd)


def kernel(x, time_steps, static, params):
    wraw, braw = _fold_lin_bn(params["raw_proj"]["lin"], params["raw_proj"]["bn"])
    wstat, bstat = _fold_lin_bn(params["static_net"]["lin"], params["static_net"]["bn"])

    g0 = params["gat"][0]
    w0 = g0["lin_w"]
    w0col = w0[:, 0][None, :]
    w0pos = w0[:, 1:].T
    pw = g0["proj"]["w"]
    pw0 = pw[:, 0][None, :]
    wppos = pw[:, 1:].T
    pb = g0["proj"]["b"][None, :]
    g1 = params["gat"][1]

    def _att_big(att):
        a1 = (jnp.eye(_HEADS, dtype=jnp.float32)[:, None, :] * att[:, :, None]).reshape(128, _HEADS)
        return jnp.kron(jnp.eye(_NODES, dtype=jnp.float32), a1)

    def _tile13(v):
        return jnp.tile(v, _NODES)[None, :]

    wg = params["fusion_gate"]["w"].T
    bg = params["fusion_gate"]["b"][None, :]

    gru1f, gru1b = params["gru"][0]
    gru2f, gru2b = params["gru"][1]

    lng = params["gru_norm"]["gamma"][None, :]
    lnb = params["gru_norm"]["beta"][None, :]
    rbn_s, rbn_c = _bn_fold(params["res"]["bn"])
    wres = params["res"]["lin"]["w"].T
    bres = params["res"]["lin"]["b"][None, :]
    wsh = params["res"]["short"]["w"].T
    bsh = params["res"]["short"]["b"][None, :]
    obn_s, obn_c = _bn_fold(params["out_bn"])
    wout = params["out_lin"]["w"].T
    bout = params["out_lin"]["b"][None, :]

    idx = time_steps.reshape(-1).astype(jnp.int32)
    pe_pad = jnp.zeros((_PE.shape[0], 128), jnp.float32).at[:, :_POS].set(jnp.asarray(_PE))
    pos = _pe_gather(pe_pad, idx)[:, :_POS].reshape(_B, _T, _POS)

    nblk = _B // _BBLK
    k1_w = [wraw, braw, wstat, bstat,
            w0col, w0pos, pw0, wppos, pb,
            _att_big(g0["att_src"]), _att_big(g0["att_dst"]),
            _tile13(g0["bias"]), _tile13(g0["ln"]["gamma"]), _tile13(g0["ln"]["beta"]),
            g1["lin_w"].T,
            _att_big(g1["att_src"]), _att_big(g1["att_dst"]),
            _tile13(g1["bias"]), _tile13(g1["ln"]["gamma"]), _tile13(g1["ln"]["beta"]),
            wg, bg,
            jnp.asarray(_SRC_EXP), jnp.asarray(_DST_EXP), jnp.asarray(_RED),
            jnp.asarray(_MASKP), jnp.asarray(_EXPC),
            jnp.asarray(_ONE128), jnp.asarray(_E13)] + [jnp.asarray(m) for m in _EXPCI]
    fused, statf = pl.pallas_call(
        _k1_body,
        grid=(nblk,),
        in_specs=[
            pl.BlockSpec((_BBLK, _T, _DYN), lambda i: (i, 0, 0)),
            pl.BlockSpec((_BBLK, _T, _POS), lambda i: (i, 0, 0)),
            pl.BlockSpec((_BBLK, _STAT), lambda i: (i, 0)),
        ] + [_full(w.shape) for w in k1_w],
        out_specs=[
            pl.BlockSpec((_T, _BBLK, 256), lambda i: (0, i, 0)),
            pl.BlockSpec((_BBLK, 64), lambda i: (i, 0)),
        ],
        out_shape=[
            jax.ShapeDtypeStruct((_T, _B, 256), jnp.float32),
            jax.ShapeDtypeStruct((_B, 64), jnp.float32),
        ],
        compiler_params=pltpu.CompilerParams(dimension_semantics=("arbitrary",)),
    )(x, pos, static, *k1_w)

    k2_w = [gru1f["w_ih"].T, gru1f["w_hh"].T, gru1f["b_ih"][None, :], gru1f["b_hh"][None, :],
            gru1b["w_ih"].T, gru1b["w_hh"].T, gru1b["b_ih"][None, :], gru1b["b_hh"][None, :]]
    f1, b1 = pl.pallas_call(
        _k2_body,
        grid=(_T,),
        in_specs=[
            pl.BlockSpec((1, _B, 256), lambda t: (t, 0, 0)),
            pl.BlockSpec((1, _B, 256), lambda t: (_T - 1 - t, 0, 0)),
            _full((_B, 64)),
        ] + [_full(w.shape) for w in k2_w],
        out_specs=[
            pl.BlockSpec((1, _B, _HID), lambda t: (t, 0, 0)),
            pl.BlockSpec((1, _B, _HID), lambda t: (_T - 1 - t, 0, 0)),
        ],
        out_shape=[
            jax.ShapeDtypeStruct((_T, _B, _HID), jnp.float32),
            jax.ShapeDtypeStruct((_T, _B, _HID), jnp.float32),
        ],
        scratch_shapes=[pltpu.VMEM((_B, _HID), jnp.float32)] * 2,
        compiler_params=pltpu.CompilerParams(dimension_semantics=("arbitrary",)),
    )(fused, fused, statf, *k2_w)

    k3_w = [gru2f["w_ih"].T, gru2f["w_hh"].T, gru2f["b_ih"][None, :], gru2f["b_hh"][None, :],
            gru2b["w_ih"].T, gru2b["w_hh"].T, gru2b["b_ih"][None, :], gru2b["b_hh"][None, :],
            lng, lnb, rbn_s, rbn_c, wres, bres, wsh, bsh, obn_s, obn_c, wout, bout]
    o = pl.pallas_call(
        _k3_body,
        grid=(_T,),
        in_specs=[
            pl.BlockSpec((1, _B, _HID), lambda t: (t, 0, 0)),
            pl.BlockSpec((1, _B, _HID), lambda t: (t, 0, 0)),
            pl.BlockSpec((1, _B, _HID), lambda t: (_T - 1, 0, 0)),
            pl.BlockSpec((1, _B, _HID), lambda t: (_T - 1, 0, 0)),
        ] + [_full(w.shape) for w in k3_w],
        out_specs=pl.BlockSpec((_B, _DYN), lambda t: (0, 0)),
        out_shape=jax.ShapeDtypeStruct((_B, _DYN), jnp.float32),
        scratch_shapes=[pltpu.VMEM((_B, _HID), jnp.float32)] * 2,
        compiler_params=pltpu.CompilerParams(dimension_semantics=("arbitrary",)),
    )(f1, b1, f1, b1, *k3_w)

    return o.reshape(_B, 1, _NODES)


# speedup vs baseline: 767.7383x; 1.0211x over previous
import functools
import math
import numpy as np
import jax
import jax.numpy as jnp
from jax.experimental import pallas as pl
from jax.experimental.pallas import tpu as pltpu
from jax.experimental.pallas import tpu_sc as plsc

_PHYSIO = [(0, 7), (0, 10), (0, 6), (0, 4), (0, 8), (0, 11), (1, 7), (2, 9),
           (3, 4), (4, 10), (5, 6), (6, 10), (8, 9), (8, 10), (10, 11), (11, 12)]
_B, _T, _DYN, _STAT, _POS, _NODES = 512, 32, 13, 7, 32, 13
_HEADS, _GAT_H = 4, 32
_HID = 256
_BBLK = 16


def _adj_np():
    a = np.zeros((_NODES, _NODES), np.float32)
    for u, v in _PHYSIO:
        a[u, v] = 1.0
        a[v, u] = 1.0
    np.fill_diagonal(a, 1.0)
    return a


def _pe_np(d_model=_POS, max_len=5000):
    pos = np.arange(max_len, dtype=np.float32)[:, None]
    div = np.exp(np.arange(0, d_model, 2).astype(np.float32) * (-math.log(10000.0) / d_model))
    pe = np.zeros((max_len, d_model), np.float32)
    pe[:, 0::2] = np.sin(pos * div)
    pe[:, 1::2] = np.cos(pos * div)
    return pe


_ADJ = _adj_np()
_PE = _pe_np()

_S = _NODES * _HEADS
_P = _NODES * _NODES * _HEADS
_F = _NODES * 128
_SRC_EXP = np.kron(np.eye(_NODES, dtype=np.float32),
                   np.tile(np.eye(_HEADS, dtype=np.float32), (1, _NODES)))
_DST_EXP = np.tile(np.eye(_S, dtype=np.float32), (1, _NODES))
_SRCDST = np.vstack([_SRC_EXP, _DST_EXP])
_RED = np.ascontiguousarray(_DST_EXP.T)
_MASKP = np.repeat(_ADJ.reshape(-1), _HEADS)[None, :].astype(np.float32)
_EXPC = np.kron(np.eye(_NODES, dtype=np.float32),
                np.repeat(np.eye(_HEADS, dtype=np.float32), 32, axis=1))
_NBRS = [[j for j in range(_NODES) if _ADJ[i, j] > 0] for i in range(_NODES)]
_EXPCI = [np.concatenate([_EXPC[:, j * 128:(j + 1) * 128] for j in _NBRS[i]], axis=1)
          for i in range(_NODES)]
_E13 = np.kron(np.eye(_NODES, dtype=np.float32), np.ones((1, 128), np.float32))
_ONE128 = np.ascontiguousarray(_E13.T) / 128.0


def _pe_gather(table, idx_flat):
    info = plsc.get_sparse_core_info()
    nw = info.num_cores * info.num_subcores
    n = idx_flat.shape[0]
    b_per_w = n // nw
    d = table.shape[1]
    mesh = plsc.VectorSubcoreMesh(core_axis_name="c", subcore_axis_name="s")

    @functools.partial(
        pl.kernel, mesh=mesh,
        out_type=jax.ShapeDtypeStruct((n, d), jnp.float32),
        scratch_types=[
            pltpu.VMEM((b_per_w,), jnp.int32),
            pltpu.VMEM((b_per_w, d), jnp.float32),
            pltpu.SemaphoreType.DMA,
        ],
    )
    def k(table_hbm, idx_hbm, out_hbm, idx_v, rows_v, sem):
        wid = jax.lax.axis_index("s") * info.num_cores + jax.lax.axis_index("c")
        base = wid * b_per_w
        pltpu.sync_copy(idx_hbm.at[pl.ds(base, b_per_w)], idx_v)
        pltpu.async_copy(table_hbm.at[idx_v], rows_v, sem).wait()
        pltpu.sync_copy(rows_v, out_hbm.at[pl.ds(base, b_per_w)])

    return k(table, idx_flat)


def _fold_lin_bn(lin, bn):
    s = bn["gamma"] * jax.lax.rsqrt(bn["var"] + 1e-5)
    w = lin["w"].T * s[None, :]
    b = (lin["b"] - bn["mean"]) * s + bn["beta"]
    return w, b[None, :]


def _bn_fold(bn):
    s = bn["gamma"] * jax.lax.rsqrt(bn["var"] + 1e-5)
    c = bn["beta"] - bn["mean"] * s
    return s[None, :], c[None, :]


def _ln_in(v, g, b):
    m = v.mean(-1, keepdims=True)
    var = ((v - m) ** 2).mean(-1, keepdims=True)
    return (v - m) * jax.lax.rsqrt(var + 1e-5) * g + b


def _erf(x):
    ax = jnp.abs(x)
    t = 1.0 / (1.0 + 0.3275911 * ax)
    poly = t * (0.254829592 + t * (-0.284496736 + t * (1.421413741 + t * (-1.453152027 + t * 1.061405429))))
    y = 1.0 - poly * jnp.exp(-ax * ax)
    return jnp.sign(x) * y


def _gelu_exact(x):
    return 0.5 * x * (1.0 + _erf(x * 0.7071067811865476))


def _attn_agg(hb, asdB, srcdst, dstexp, red, maskp, expcis):
    asd = jnp.dot(hb, asdB, preferred_element_type=jnp.float32)
    e = jnp.dot(asd, srcdst, preferred_element_type=jnp.float32)
    e = jnp.where(e >= 0, e, 0.2 * e)
    e = jnp.where(maskp > 0, e, -1e30)
    m = jnp.max(e, axis=-1, keepdims=True)
    ee = jnp.exp(e - m) * maskp
    den = jnp.dot(ee, red, preferred_element_type=jnp.float32)
    alpha = ee * jnp.dot(1.0 / (den + 1e-16), dstexp,
                         preferred_element_type=jnp.float32)
    parts = [None] * _NODES
    for i in range(_NODES):
        aexp = jnp.dot(alpha[:, i * _S:(i + 1) * _S], expcis[i],
                       preferred_element_type=jnp.float32)
        hs = hb[:, i * 128:(i + 1) * 128]
        for k, j in enumerate(_NBRS[i]):
            c = aexp[:, k * 128:(k + 1) * 128] * hs
            parts[j] = c if parts[j] is None else parts[j] + c
    return jnp.concatenate(parts, axis=1)


def _ln_big(v, one128, e13, gtile, btile):
    m13 = jnp.dot(v, one128, preferred_element_type=jnp.float32)
    d = v - jnp.dot(m13, e13, preferred_element_type=jnp.float32)
    v13 = jnp.dot(d * d, one128, preferred_element_type=jnp.float32)
    rstd = jax.lax.rsqrt(v13 + 1e-5)
    return d * jnp.dot(rstd, e13, preferred_element_type=jnp.float32) * gtile + btile


def _k1_body(x_ref, pos_ref, static_ref,
             wraw_ref, braw_ref, wstat_ref, bstat_ref,
             w0col_ref, w0pos_ref, pw0_ref, wppos_ref, pb_ref,
             asd0_ref, bias0_ref, ln0g_ref, ln0b_ref,
             w1t_ref, asd1_ref, bias1_ref, ln1g_ref, ln1b_ref,
             wg_ref, bg_ref,
             srcdst_ref, dstexp_ref, red_ref, maskp_ref,
             one128_ref, e13_ref,
             *rest):
    expci = [r[...] for r in rest[:_NODES]]
    fused_out_ref, stat_out_ref = rest[_NODES], rest[_NODES + 1]
    g = _BBLK * _T
    xb = x_ref[...].reshape(g, _DYN)
    posb = pos_ref[...].reshape(g, _POS)
    srcdst, dstexp, red = srcdst_ref[...], dstexp_ref[...], red_ref[...]
    maskp = maskp_ref[...]
    one128, e13 = one128_ref[...], e13_ref[...]

    p0 = jnp.dot(posb, w0pos_ref[...], preferred_element_type=jnp.float32)
    w0col = w0col_ref[...]
    hbig = jnp.concatenate([xb[:, n:n + 1] * w0col + p0 for n in range(_NODES)], axis=1)
    rp = jnp.dot(posb, wppos_ref[...], preferred_element_type=jnp.float32) + pb_ref[...]
    pw0 = pw0_ref[...]
    resbig = jnp.concatenate([xb[:, n:n + 1] * pw0 + rp for n in range(_NODES)], axis=1)

    agg0 = _attn_agg(hbig, asd0_ref[...], srcdst, dstexp, red, maskp, expci)
    agg0 = agg0 + bias0_ref[...]
    h1 = jax.nn.relu(_ln_big(agg0, one128, e13, ln0g_ref[...], ln0b_ref[...]) + resbig)

    w1t = w1t_ref[...]
    h14 = jnp.concatenate(
        [jnp.dot(h1[:, n * 128:(n + 1) * 128], w1t, preferred_element_type=jnp.float32)
         for n in range(_NODES)], axis=1)
    agg1 = _attn_agg(h14, asd1_ref[...], srcdst, dstexp, red, maskp, expci)
    agg1 = agg1 + bias1_ref[...]
    h2 = jax.nn.relu(_ln_big(agg1, one128, e13, ln1g_ref[...], ln1b_ref[...]) + h1)

    gat_seq = h2[:, 0:128]
    for n in range(1, _NODES):
        gat_seq = gat_seq + h2[:, n * 128:(n + 1) * 128]
    gat_seq = gat_seq * (1.0 / _NODES)

    raw = jnp.dot(xb, wraw_ref[...], preferred_element_type=jnp.float32) + braw_ref[...]
    raw = _gelu_exact(raw)
    ff = jnp.concatenate([raw, gat_seq], axis=-1)
    gate = jax.nn.sigmoid(jnp.dot(ff, wg_ref[...], preferred_element_type=jnp.float32) + bg_ref[...])
    fused = gate * ff
    fused_out_ref[...] = fused.reshape(_BBLK, _T, 256).transpose(1, 0, 2)

    st = jnp.dot(static_ref[...], wstat_ref[...], preferred_element_type=jnp.float32) + bstat_ref[...]
    stat_out_ref[...] = jnp.where(st >= 0, st, 0.01 * st)


def _gru_step(xt, h, wih, whh, bih, bhh):
    gi = jnp.dot(xt, wih, preferred_element_type=jnp.float32) + bih
    gh = jnp.dot(h, whh, preferred_element_type=jnp.float32) + bhh
    r = jax.nn.sigmoid(gi[:, :_HID] + gh[:, :_HID])
    z = jax.nn.sigmoid(gi[:, _HID:2 * _HID] + gh[:, _HID:2 * _HID])
    n = jnp.tanh(gi[:, 2 * _HID:] + r * gh[:, 2 * _HID:])
    return (1.0 - z) * n + z * h


def _k2_body(fused_f_ref, fused_b_ref, stat_ref,
             wih_f_ref, whh_f_ref, bih_f_ref, bhh_f_ref,
             wih_b_ref, whh_b_ref, bih_b_ref, bhh_b_ref,
             f1_ref, b1_ref, hf_scr, hb_scr):
    t = pl.program_id(0)

    @pl.when(t == 0)
    def _():
        hf_scr[...] = jnp.zeros_like(hf_scr)
        hb_scr[...] = jnp.zeros_like(hb_scr)

    stat = stat_ref[...]
    xf = jnp.concatenate([fused_f_ref[0], stat], axis=-1)
    hf = _gru_step(xf, hf_scr[...], wih_f_ref[...], whh_f_ref[...], bih_f_ref[...], bhh_f_ref[...])
    hf_scr[...] = hf
    f1_ref[0] = hf

    xb = jnp.concatenate([fused_b_ref[0], stat], axis=-1)
    hb = _gru_step(xb, hb_scr[...], wih_b_ref[...], whh_b_ref[...], bih_b_ref[...], bhh_b_ref[...])
    hb_scr[...] = hb
    b1_ref[0] = hb


def _k3_body(f1_ref, b1_ref, f1last_ref, b1last_ref,
             wih_f_ref, whh_f_ref, bih_f_ref, bhh_f_ref,
             wih_b_ref, whh_b_ref, bih_b_ref, bhh_b_ref,
             lng_ref, lnb_ref, rbn_s_ref, rbn_c_ref,
             wres_ref, bres_ref, wsh_ref, bsh_ref,
             obn_s_ref, obn_c_ref, wout_ref, bout_ref,
             o_ref, hf_scr, hb_scr):
    t = pl.program_id(0)

    @pl.when(t == 0)
    def _():
        hf_scr[...] = jnp.zeros_like(hf_scr)
        xlast = jnp.concatenate([f1last_ref[0], b1last_ref[0]], axis=-1)
        gi = jnp.dot(xlast, wih_b_ref[...], preferred_element_type=jnp.float32) + bih_b_ref[...]
        bhh = bhh_b_ref[...]
        r = jax.nn.sigmoid(gi[:, :_HID] + bhh[:, :_HID])
        z = jax.nn.sigmoid(gi[:, _HID:2 * _HID] + bhh[:, _HID:2 * _HID])
        n = jnp.tanh(gi[:, 2 * _HID:] + r * bhh[:, 2 * _HID:])
        hb_scr[...] = (1.0 - z) * n

    xt = jnp.concatenate([f1_ref[0], b1_ref[0]], axis=-1)
    hf = _gru_step(xt, hf_scr[...], wih_f_ref[...], whh_f_ref[...], bih_f_ref[...], bhh_f_ref[...])
    hf_scr[...] = hf

    @pl.when(t == _T - 1)
    def _():
        feat = jnp.concatenate([hf, hb_scr[...]], axis=-1)
        feat = _ln_in(feat, lng_ref[...], lnb_ref[...])
        v = feat * rbn_s_ref[...] + rbn_c_ref[...]
        v = jax.nn.relu(v)
        o1 = (jnp.dot(v, wres_ref[...], preferred_element_type=jnp.float32) + bres_ref[...]
              + jnp.dot(feat, wsh_ref[...], preferred_element_type=jnp.float32) + bsh_ref[...])
        o1 = o1 * obn_s_ref[...] + obn_c_ref[...]
        o1 = jnp.where(o1 >= 0, o1, 0.01 * o1)
        o_ref[...] = jnp.dot(o1, wout_ref[...], preferred_element_type=jnp.float32) + bout_ref[...]


def _full(shape):
    nd = len(shape)
    return pl.BlockSpec(shape, lambda *a: (0,) * nd)


def kernel(x, time_steps, static, params):
    wraw, braw = _fold_lin_bn(params["raw_proj"]["lin"], params["raw_proj"]["bn"])
    wstat, bstat = _fold_lin_bn(params["static_net"]["lin"], params["static_net"]["bn"])

    g0 = params["gat"][0]
    w0 = g0["lin_w"]
    w0col = w0[:, 0][None, :]
    w0pos = w0[:, 1:].T
    pw = g0["proj"]["w"]
    pw0 = pw[:, 0][None, :]
    wppos = pw[:, 1:].T
    pb = g0["proj"]["b"][None, :]
    g1 = params["gat"][1]

    def _att_big(att):
        a1 = (jnp.eye(_HEADS, dtype=jnp.float32)[:, None, :] * att[:, :, None]).reshape(128, _HEADS)
        return jnp.kron(jnp.eye(_NODES, dtype=jnp.float32), a1)

    def _tile13(v):
        return jnp.tile(v, _NODES)[None, :]

    wg = params["fusion_gate"]["w"].T
    bg = params["fusion_gate"]["b"][None, :]

    gru1f, gru1b = params["gru"][0]
    gru2f, gru2b = params["gru"][1]

    lng = params["gru_norm"]["gamma"][None, :]
    lnb = params["gru_norm"]["beta"][None, :]
    rbn_s, rbn_c = _bn_fold(params["res"]["bn"])
    wres = params["res"]["lin"]["w"].T
    bres = params["res"]["lin"]["b"][None, :]
    wsh = params["res"]["short"]["w"].T
    bsh = params["res"]["short"]["b"][None, :]
    obn_s, obn_c = _bn_fold(params["out_bn"])
    wout = params["out_lin"]["w"].T
    bout = params["out_lin"]["b"][None, :]

    idx = time_steps.reshape(-1).astype(jnp.int32)
    pe_pad = jnp.zeros((_PE.shape[0], 128), jnp.float32).at[:, :_POS].set(jnp.asarray(_PE))
    pos = _pe_gather(pe_pad, idx)[:, :_POS].reshape(_B, _T, _POS)

    nblk = _B // _BBLK
    k1_w = [wraw, braw, wstat, bstat,
            w0col, w0pos, pw0, wppos, pb,
            jnp.concatenate([_att_big(g0["att_src"]), _att_big(g0["att_dst"])], axis=1),
            _tile13(g0["bias"]), _tile13(g0["ln"]["gamma"]), _tile13(g0["ln"]["beta"]),
            g1["lin_w"].T,
            jnp.concatenate([_att_big(g1["att_src"]), _att_big(g1["att_dst"])], axis=1),
            _tile13(g1["bias"]), _tile13(g1["ln"]["gamma"]), _tile13(g1["ln"]["beta"]),
            wg, bg,
            jnp.asarray(_SRCDST), jnp.asarray(_DST_EXP), jnp.asarray(_RED),
            jnp.asarray(_MASKP),
            jnp.asarray(_ONE128), jnp.asarray(_E13)] + [jnp.asarray(m) for m in _EXPCI]
    fused, statf = pl.pallas_call(
        _k1_body,
        grid=(nblk,),
        in_specs=[
            pl.BlockSpec((_BBLK, _T, _DYN), lambda i: (i, 0, 0)),
            pl.BlockSpec((_BBLK, _T, _POS), lambda i: (i, 0, 0)),
            pl.BlockSpec((_BBLK, _STAT), lambda i: (i, 0)),
        ] + [_full(w.shape) for w in k1_w],
        out_specs=[
            pl.BlockSpec((_T, _BBLK, 256), lambda i: (0, i, 0)),
            pl.BlockSpec((_BBLK, 64), lambda i: (i, 0)),
        ],
        out_shape=[
            jax.ShapeDtypeStruct((_T, _B, 256), jnp.float32),
            jax.ShapeDtypeStruct((_B, 64), jnp.float32),
        ],
        compiler_params=pltpu.CompilerParams(dimension_semantics=("arbitrary",)),
    )(x, pos, static, *k1_w)

    k2_w = [gru1f["w_ih"].T, gru1f["w_hh"].T, gru1f["b_ih"][None, :], gru1f["b_hh"][None, :],
            gru1b["w_ih"].T, gru1b["w_hh"].T, gru1b["b_ih"][None, :], gru1b["b_hh"][None, :]]
    f1, b1 = pl.pallas_call(
        _k2_body,
        grid=(_T,),
        in_specs=[
            pl.BlockSpec((1, _B, 256), lambda t: (t, 0, 0)),
            pl.BlockSpec((1, _B, 256), lambda t: (_T - 1 - t, 0, 0)),
            _full((_B, 64)),
        ] + [_full(w.shape) for w in k2_w],
        out_specs=[
            pl.BlockSpec((1, _B, _HID), lambda t: (t, 0, 0)),
            pl.BlockSpec((1, _B, _HID), lambda t: (_T - 1 - t, 0, 0)),
        ],
        out_shape=[
            jax.ShapeDtypeStruct((_T, _B, _HID), jnp.float32),
            jax.ShapeDtypeStruct((_T, _B, _HID), jnp.float32),
        ],
        scratch_shapes=[pltpu.VMEM((_B, _HID), jnp.float32)] * 2,
        compiler_params=pltpu.CompilerParams(dimension_semantics=("arbitrary",)),
    )(fused, fused, statf, *k2_w)

    k3_w = [gru2f["w_ih"].T, gru2f["w_hh"].T, gru2f["b_ih"][None, :], gru2f["b_hh"][None, :],
            gru2b["w_ih"].T, gru2b["w_hh"].T, gru2b["b_ih"][None, :], gru2b["b_hh"][None, :],
            lng, lnb, rbn_s, rbn_c, wres, bres, wsh, bsh, obn_s, obn_c, wout, bout]
    o = pl.pallas_call(
        _k3_body,
        grid=(_T,),
        in_specs=[
            pl.BlockSpec((1, _B, _HID), lambda t: (t, 0, 0)),
            pl.BlockSpec((1, _B, _HID), lambda t: (t, 0, 0)),
            pl.BlockSpec((1, _B, _HID), lambda t: (_T - 1, 0, 0)),
            pl.BlockSpec((1, _B, _HID), lambda t: (_T - 1, 0, 0)),
        ] + [_full(w.shape) for w in k3_w],
        out_specs=pl.BlockSpec((_B, _DYN), lambda t: (0, 0)),
        out_shape=jax.ShapeDtypeStruct((_B, _DYN), jnp.float32),
        scratch_shapes=[pltpu.VMEM((_B, _HID), jnp.float32)] * 2,
        compiler_params=pltpu.CompilerParams(dimension_semantics=("arbitrary",)),
    )(f1, b1, f1, b1, *k3_w)

    return o.reshape(_B, 1, _NODES)


# speedup vs baseline: 785.3179x; 1.0229x over previous
import functools
import math
import numpy as np
import jax
import jax.numpy as jnp
from jax.experimental import pallas as pl
from jax.experimental.pallas import tpu as pltpu
from jax.experimental.pallas import tpu_sc as plsc

_PHYSIO = [(0, 7), (0, 10), (0, 6), (0, 4), (0, 8), (0, 11), (1, 7), (2, 9),
           (3, 4), (4, 10), (5, 6), (6, 10), (8, 9), (8, 10), (10, 11), (11, 12)]
_B, _T, _DYN, _STAT, _POS, _NODES = 512, 32, 13, 7, 32, 13
_HEADS, _GAT_H = 4, 32
_HID = 256
_BBLK = 32


def _adj_np():
    a = np.zeros((_NODES, _NODES), np.float32)
    for u, v in _PHYSIO:
        a[u, v] = 1.0
        a[v, u] = 1.0
    np.fill_diagonal(a, 1.0)
    return a


def _pe_np(d_model=_POS, max_len=5000):
    pos = np.arange(max_len, dtype=np.float32)[:, None]
    div = np.exp(np.arange(0, d_model, 2).astype(np.float32) * (-math.log(10000.0) / d_model))
    pe = np.zeros((max_len, d_model), np.float32)
    pe[:, 0::2] = np.sin(pos * div)
    pe[:, 1::2] = np.cos(pos * div)
    return pe


_ADJ = _adj_np()
_PE = _pe_np()

_S = _NODES * _HEADS
_P = _NODES * _NODES * _HEADS
_F = _NODES * 128
_SRC_EXP = np.kron(np.eye(_NODES, dtype=np.float32),
                   np.tile(np.eye(_HEADS, dtype=np.float32), (1, _NODES)))
_DST_EXP = np.tile(np.eye(_S, dtype=np.float32), (1, _NODES))
_SRCDST = np.vstack([_SRC_EXP, _DST_EXP])
_RED = np.ascontiguousarray(_DST_EXP.T)
_MASKP = np.repeat(_ADJ.reshape(-1), _HEADS)[None, :].astype(np.float32)
_EXPC = np.kron(np.eye(_NODES, dtype=np.float32),
                np.repeat(np.eye(_HEADS, dtype=np.float32), 32, axis=1))
_NBRS = [[j for j in range(_NODES) if _ADJ[i, j] > 0] for i in range(_NODES)]
_EXPCI = [np.concatenate([_EXPC[:, j * 128:(j + 1) * 128] for j in _NBRS[i]], axis=1)
          for i in range(_NODES)]
_E13 = np.kron(np.eye(_NODES, dtype=np.float32), np.ones((1, 128), np.float32))
_ONE128 = np.ascontiguousarray(_E13.T) / 128.0


def _pe_gather(table, idx_flat):
    info = plsc.get_sparse_core_info()
    nw = info.num_cores * info.num_subcores
    n = idx_flat.shape[0]
    b_per_w = n // nw
    d = table.shape[1]
    mesh = plsc.VectorSubcoreMesh(core_axis_name="c", subcore_axis_name="s")

    @functools.partial(
        pl.kernel, mesh=mesh,
        out_type=jax.ShapeDtypeStruct((n, d), jnp.float32),
        scratch_types=[
            pltpu.VMEM((b_per_w,), jnp.int32),
            pltpu.VMEM((b_per_w, d), jnp.float32),
            pltpu.SemaphoreType.DMA,
        ],
    )
    def k(table_hbm, idx_hbm, out_hbm, idx_v, rows_v, sem):
        wid = jax.lax.axis_index("s") * info.num_cores + jax.lax.axis_index("c")
        base = wid * b_per_w
        pltpu.sync_copy(idx_hbm.at[pl.ds(base, b_per_w)], idx_v)
        pltpu.async_copy(table_hbm.at[idx_v], rows_v, sem).wait()
        pltpu.sync_copy(rows_v, out_hbm.at[pl.ds(base, b_per_w)])

    return k(table, idx_flat)


def _fold_lin_bn(lin, bn):
    s = bn["gamma"] * jax.lax.rsqrt(bn["var"] + 1e-5)
    w = lin["w"].T * s[None, :]
    b = (lin["b"] - bn["mean"]) * s + bn["beta"]
    return w, b[None, :]


def _bn_fold(bn):
    s = bn["gamma"] * jax.lax.rsqrt(bn["var"] + 1e-5)
    c = bn["beta"] - bn["mean"] * s
    return s[None, :], c[None, :]


def _ln_in(v, g, b):
    m = v.mean(-1, keepdims=True)
    var = ((v - m) ** 2).mean(-1, keepdims=True)
    return (v - m) * jax.lax.rsqrt(var + 1e-5) * g + b


def _erf(x):
    ax = jnp.abs(x)
    t = 1.0 / (1.0 + 0.3275911 * ax)
    poly = t * (0.254829592 + t * (-0.284496736 + t * (1.421413741 + t * (-1.453152027 + t * 1.061405429))))
    y = 1.0 - poly * jnp.exp(-ax * ax)
    return jnp.sign(x) * y


def _gelu_exact(x):
    return 0.5 * x * (1.0 + _erf(x * 0.7071067811865476))


def _attn_agg(hb, asdB, srcdst, dstexp, red, maskp, expcis):
    asd = jnp.dot(hb, asdB, preferred_element_type=jnp.float32)
    e = jnp.dot(asd, srcdst, preferred_element_type=jnp.float32)
    e = jnp.where(e >= 0, e, 0.2 * e)
    e = jnp.where(maskp > 0, e, -1e30)
    m = jnp.max(e, axis=-1, keepdims=True)
    ee = jnp.exp(e - m) * maskp
    den = jnp.dot(ee, red, preferred_element_type=jnp.float32)
    alpha = ee * jnp.dot(1.0 / (den + 1e-16), dstexp,
                         preferred_element_type=jnp.float32)
    parts = [None] * _NODES
    for i in range(_NODES):
        aexp = jnp.dot(alpha[:, i * _S:(i + 1) * _S], expcis[i],
                       preferred_element_type=jnp.float32)
        hs = hb[:, i * 128:(i + 1) * 128]
        for k, j in enumerate(_NBRS[i]):
            c = aexp[:, k * 128:(k + 1) * 128] * hs
            parts[j] = c if parts[j] is None else parts[j] + c
    return jnp.concatenate(parts, axis=1)


def _ln_big(v, one128, e13, gtile, btile):
    m13 = jnp.dot(v, one128, preferred_element_type=jnp.float32)
    d = v - jnp.dot(m13, e13, preferred_element_type=jnp.float32)
    v13 = jnp.dot(d * d, one128, preferred_element_type=jnp.float32)
    rstd = jax.lax.rsqrt(v13 + 1e-5)
    return d * jnp.dot(rstd, e13, preferred_element_type=jnp.float32) * gtile + btile


def _k1_body(x_ref, pos_ref, static_ref,
             wraw_ref, braw_ref, wstat_ref, bstat_ref,
             w0col_ref, w0pos_ref, pw0_ref, wppos_ref, pb_ref,
             asd0_ref, bias0_ref, ln0g_ref, ln0b_ref,
             w1t_ref, asd1_ref, bias1_ref, ln1g_ref, ln1b_ref,
             wg_ref, bg_ref,
             srcdst_ref, dstexp_ref, red_ref, maskp_ref,
             one128_ref, e13_ref,
             *rest):
    expci = [r[...] for r in rest[:_NODES]]
    fused_out_ref, stat_out_ref = rest[_NODES], rest[_NODES + 1]
    g = _BBLK * _T
    xb = x_ref[...].reshape(g, _DYN)
    posb = pos_ref[...].reshape(g, _POS)
    srcdst, dstexp, red = srcdst_ref[...], dstexp_ref[...], red_ref[...]
    maskp = maskp_ref[...]
    one128, e13 = one128_ref[...], e13_ref[...]

    p0 = jnp.dot(posb, w0pos_ref[...], preferred_element_type=jnp.float32)
    w0col = w0col_ref[...]
    hbig = jnp.concatenate([xb[:, n:n + 1] * w0col + p0 for n in range(_NODES)], axis=1)
    rp = jnp.dot(posb, wppos_ref[...], preferred_element_type=jnp.float32) + pb_ref[...]
    pw0 = pw0_ref[...]
    resbig = jnp.concatenate([xb[:, n:n + 1] * pw0 + rp for n in range(_NODES)], axis=1)

    agg0 = _attn_agg(hbig, asd0_ref[...], srcdst, dstexp, red, maskp, expci)
    agg0 = agg0 + bias0_ref[...]
    h1 = jax.nn.relu(_ln_big(agg0, one128, e13, ln0g_ref[...], ln0b_ref[...]) + resbig)

    w1t = w1t_ref[...]
    h14 = jnp.concatenate(
        [jnp.dot(h1[:, n * 128:(n + 1) * 128], w1t, preferred_element_type=jnp.float32)
         for n in range(_NODES)], axis=1)
    agg1 = _attn_agg(h14, asd1_ref[...], srcdst, dstexp, red, maskp, expci)
    agg1 = agg1 + bias1_ref[...]
    h2 = jax.nn.relu(_ln_big(agg1, one128, e13, ln1g_ref[...], ln1b_ref[...]) + h1)

    gat_seq = h2[:, 0:128]
    for n in range(1, _NODES):
        gat_seq = gat_seq + h2[:, n * 128:(n + 1) * 128]
    gat_seq = gat_seq * (1.0 / _NODES)

    raw = jnp.dot(xb, wraw_ref[...], preferred_element_type=jnp.float32) + braw_ref[...]
    raw = _gelu_exact(raw)
    ff = jnp.concatenate([raw, gat_seq], axis=-1)
    gate = jax.nn.sigmoid(jnp.dot(ff, wg_ref[...], preferred_element_type=jnp.float32) + bg_ref[...])
    fused = gate * ff
    fused_out_ref[...] = fused.reshape(_BBLK, _T, 256).transpose(1, 0, 2)

    st = jnp.dot(static_ref[...], wstat_ref[...], preferred_element_type=jnp.float32) + bstat_ref[...]
    stat_out_ref[...] = jnp.where(st >= 0, st, 0.01 * st)


def _gru_step(xt, h, wih, whh, bih, bhh):
    gi = jnp.dot(xt, wih, preferred_element_type=jnp.float32) + bih
    gh = jnp.dot(h, whh, preferred_element_type=jnp.float32) + bhh
    r = jax.nn.sigmoid(gi[:, :_HID] + gh[:, :_HID])
    z = jax.nn.sigmoid(gi[:, _HID:2 * _HID] + gh[:, _HID:2 * _HID])
    n = jnp.tanh(gi[:, 2 * _HID:] + r * gh[:, 2 * _HID:])
    return (1.0 - z) * n + z * h


def _k2_body(fused_f_ref, fused_b_ref, stat_ref,
             wih_f_ref, whh_f_ref, bih_f_ref, bhh_f_ref,
             wih_b_ref, whh_b_ref, bih_b_ref, bhh_b_ref,
             f1_ref, b1_ref, hf_scr, hb_scr):
    t = pl.program_id(0)

    @pl.when(t == 0)
    def _():
        hf_scr[...] = jnp.zeros_like(hf_scr)
        hb_scr[...] = jnp.zeros_like(hb_scr)

    stat = stat_ref[...]
    xf = jnp.concatenate([fused_f_ref[0], stat], axis=-1)
    hf = _gru_step(xf, hf_scr[...], wih_f_ref[...], whh_f_ref[...], bih_f_ref[...], bhh_f_ref[...])
    hf_scr[...] = hf
    f1_ref[0] = hf

    xb = jnp.concatenate([fused_b_ref[0], stat], axis=-1)
    hb = _gru_step(xb, hb_scr[...], wih_b_ref[...], whh_b_ref[...], bih_b_ref[...], bhh_b_ref[...])
    hb_scr[...] = hb
    b1_ref[0] = hb


def _k3_body(f1_ref, b1_ref, f1last_ref, b1last_ref,
             wih_f_ref, whh_f_ref, bih_f_ref, bhh_f_ref,
             wih_b_ref, whh_b_ref, bih_b_ref, bhh_b_ref,
             lng_ref, lnb_ref, rbn_s_ref, rbn_c_ref,
             wres_ref, bres_ref, wsh_ref, bsh_ref,
             obn_s_ref, obn_c_ref, wout_ref, bout_ref,
             o_ref, hf_scr, hb_scr):
    t = pl.program_id(0)

    @pl.when(t == 0)
    def _():
        hf_scr[...] = jnp.zeros_like(hf_scr)
        xlast = jnp.concatenate([f1last_ref[0], b1last_ref[0]], axis=-1)
        gi = jnp.dot(xlast, wih_b_ref[...], preferred_element_type=jnp.float32) + bih_b_ref[...]
        bhh = bhh_b_ref[...]
        r = jax.nn.sigmoid(gi[:, :_HID] + bhh[:, :_HID])
        z = jax.nn.sigmoid(gi[:, _HID:2 * _HID] + bhh[:, _HID:2 * _HID])
        n = jnp.tanh(gi[:, 2 * _HID:] + r * bhh[:, 2 * _HID:])
        hb_scr[...] = (1.0 - z) * n

    xt = jnp.concatenate([f1_ref[0], b1_ref[0]], axis=-1)
    hf = _gru_step(xt, hf_scr[...], wih_f_ref[...], whh_f_ref[...], bih_f_ref[...], bhh_f_ref[...])
    hf_scr[...] = hf

    @pl.when(t == _T - 1)
    def _():
        feat = jnp.concatenate([hf, hb_scr[...]], axis=-1)
        feat = _ln_in(feat, lng_ref[...], lnb_ref[...])
        v = feat * rbn_s_ref[...] + rbn_c_ref[...]
        v = jax.nn.relu(v)
        o1 = (jnp.dot(v, wres_ref[...], preferred_element_type=jnp.float32) + bres_ref[...]
              + jnp.dot(feat, wsh_ref[...], preferred_element_type=jnp.float32) + bsh_ref[...])
        o1 = o1 * obn_s_ref[...] + obn_c_ref[...]
        o1 = jnp.where(o1 >= 0, o1, 0.01 * o1)
        o_ref[...] = jnp.dot(o1, wout_ref[...], preferred_element_type=jnp.float32) + bout_ref[...]


def _full(shape):
    nd = len(shape)
    return pl.BlockSpec(shape, lambda *a: (0,) * nd)


def kernel(x, time_steps, static, params):
    wraw, braw = _fold_lin_bn(params["raw_proj"]["lin"], params["raw_proj"]["bn"])
    wstat, bstat = _fold_lin_bn(params["static_net"]["lin"], params["static_net"]["bn"])

    g0 = params["gat"][0]
    w0 = g0["lin_w"]
    w0col = w0[:, 0][None, :]
    w0pos = w0[:, 1:].T
    pw = g0["proj"]["w"]
    pw0 = pw[:, 0][None, :]
    wppos = pw[:, 1:].T
    pb = g0["proj"]["b"][None, :]
    g1 = params["gat"][1]

    def _att_big(att):
        a1 = (jnp.eye(_HEADS, dtype=jnp.float32)[:, None, :] * att[:, :, None]).reshape(128, _HEADS)
        return jnp.kron(jnp.eye(_NODES, dtype=jnp.float32), a1)

    def _tile13(v):
        return jnp.tile(v, _NODES)[None, :]

    wg = params["fusion_gate"]["w"].T
    bg = params["fusion_gate"]["b"][None, :]

    gru1f, gru1b = params["gru"][0]
    gru2f, gru2b = params["gru"][1]

    lng = params["gru_norm"]["gamma"][None, :]
    lnb = params["gru_norm"]["beta"][None, :]
    rbn_s, rbn_c = _bn_fold(params["res"]["bn"])
    wres = params["res"]["lin"]["w"].T
    bres = params["res"]["lin"]["b"][None, :]
    wsh = params["res"]["short"]["w"].T
    bsh = params["res"]["short"]["b"][None, :]
    obn_s, obn_c = _bn_fold(params["out_bn"])
    wout = params["out_lin"]["w"].T
    bout = params["out_lin"]["b"][None, :]

    idx = time_steps.reshape(-1).astype(jnp.int32)
    pe_pad = jnp.zeros((_PE.shape[0], 128), jnp.float32).at[:, :_POS].set(jnp.asarray(_PE))
    pos = _pe_gather(pe_pad, idx)[:, :_POS].reshape(_B, _T, _POS)

    nblk = _B // _BBLK
    k1_w = [wraw, braw, wstat, bstat,
            w0col, w0pos, pw0, wppos, pb,
            jnp.concatenate([_att_big(g0["att_src"]), _att_big(g0["att_dst"])], axis=1),
            _tile13(g0["bias"]), _tile13(g0["ln"]["gamma"]), _tile13(g0["ln"]["beta"]),
            g1["lin_w"].T,
            jnp.concatenate([_att_big(g1["att_src"]), _att_big(g1["att_dst"])], axis=1),
            _tile13(g1["bias"]), _tile13(g1["ln"]["gamma"]), _tile13(g1["ln"]["beta"]),
            wg, bg,
            jnp.asarray(_SRCDST), jnp.asarray(_DST_EXP), jnp.asarray(_RED),
            jnp.asarray(_MASKP),
            jnp.asarray(_ONE128), jnp.asarray(_E13)] + [jnp.asarray(m) for m in _EXPCI]
    fused, statf = pl.pallas_call(
        _k1_body,
        grid=(nblk,),
        in_specs=[
            pl.BlockSpec((_BBLK, _T, _DYN), lambda i: (i, 0, 0)),
            pl.BlockSpec((_BBLK, _T, _POS), lambda i: (i, 0, 0)),
            pl.BlockSpec((_BBLK, _STAT), lambda i: (i, 0)),
        ] + [_full(w.shape) for w in k1_w],
        out_specs=[
            pl.BlockSpec((_T, _BBLK, 256), lambda i: (0, i, 0)),
            pl.BlockSpec((_BBLK, 64), lambda i: (i, 0)),
        ],
        out_shape=[
            jax.ShapeDtypeStruct((_T, _B, 256), jnp.float32),
            jax.ShapeDtypeStruct((_B, 64), jnp.float32),
        ],
        compiler_params=pltpu.CompilerParams(dimension_semantics=("arbitrary",)),
    )(x, pos, static, *k1_w)

    k2_w = [gru1f["w_ih"].T, gru1f["w_hh"].T, gru1f["b_ih"][None, :], gru1f["b_hh"][None, :],
            gru1b["w_ih"].T, gru1b["w_hh"].T, gru1b["b_ih"][None, :], gru1b["b_hh"][None, :]]
    f1, b1 = pl.pallas_call(
        _k2_body,
        grid=(_T,),
        in_specs=[
            pl.BlockSpec((1, _B, 256), lambda t: (t, 0, 0)),
            pl.BlockSpec((1, _B, 256), lambda t: (_T - 1 - t, 0, 0)),
            _full((_B, 64)),
        ] + [_full(w.shape) for w in k2_w],
        out_specs=[
            pl.BlockSpec((1, _B, _HID), lambda t: (t, 0, 0)),
            pl.BlockSpec((1, _B, _HID), lambda t: (_T - 1 - t, 0, 0)),
        ],
        out_shape=[
            jax.ShapeDtypeStruct((_T, _B, _HID), jnp.float32),
            jax.ShapeDtypeStruct((_T, _B, _HID), jnp.float32),
        ],
        scratch_shapes=[pltpu.VMEM((_B, _HID), jnp.float32)] * 2,
        compiler_params=pltpu.CompilerParams(dimension_semantics=("arbitrary",)),
    )(fused, fused, statf, *k2_w)

    k3_w = [gru2f["w_ih"].T, gru2f["w_hh"].T, gru2f["b_ih"][None, :], gru2f["b_hh"][None, :],
            gru2b["w_ih"].T, gru2b["w_hh"].T, gru2b["b_ih"][None, :], gru2b["b_hh"][None, :],
            lng, lnb, rbn_s, rbn_c, wres, bres, wsh, bsh, obn_s, obn_c, wout, bout]
    o = pl.pallas_call(
        _k3_body,
        grid=(_T,),
        in_specs=[
            pl.BlockSpec((1, _B, _HID), lambda t: (t, 0, 0)),
            pl.BlockSpec((1, _B, _HID), lambda t: (t, 0, 0)),
            pl.BlockSpec((1, _B, _HID), lambda t: (_T - 1, 0, 0)),
            pl.BlockSpec((1, _B, _HID), lambda t: (_T - 1, 0, 0)),
        ] + [_full(w.shape) for w in k3_w],
        out_specs=pl.BlockSpec((_B, _DYN), lambda t: (0, 0)),
        out_shape=jax.ShapeDtypeStruct((_B, _DYN), jnp.float32),
        scratch_shapes=[pltpu.VMEM((_B, _HID), jnp.float32)] * 2,
        compiler_params=pltpu.CompilerParams(dimension_semantics=("arbitrary",)),
    )(f1, b1, f1, b1, *k3_w)

    return o.reshape(_B, 1, _NODES)
